# baseline (speedup 1.0000x reference)
import functools

import jax
import jax.numpy as jnp
from jax import lax
from jax.experimental import pallas as pl
from jax.experimental.pallas import tpu as pltpu
from jax.experimental.pallas import tpu_sc as plsc

N = 10000
NPAD = 10240
E = 160000
ETOT = 170000
EPAD = 170240
H = 128
W = 128
BN = 1280
BE = 2128
NSUB = 16
ROWS_PER_SUB = NPAD // NSUB

@functools.cache
def _sc_mesh():
    return plsc.VectorSubcoreMesh(core_axis_name="core",
                                  subcore_axis_name="subcore")


def _mm(a, b):
    return jax.lax.dot_general(
        a, b, (((1,), (0,)), ((), ())),
        precision=jax.lax.Precision.HIGHEST,
        preferred_element_type=jnp.float32)


HI_MASK = -65536


def _pack_cols(v):
    lo = v[:, :64].astype(jnp.bfloat16).astype(jnp.float32)
    hi = v[:, 64:].astype(jnp.bfloat16).astype(jnp.float32)
    lo_i = jax.lax.bitcast_convert_type(lo, jnp.int32)
    hi_i = jax.lax.bitcast_convert_type(hi, jnp.int32)
    return jax.lax.shift_right_logical(lo_i, 16) | (hi_i & jnp.int32(HI_MASK))


def _unpack_lo(p):
    return jax.lax.bitcast_convert_type(jax.lax.shift_left(p, 16), jnp.float32)


def _unpack_hi(p):
    return jax.lax.bitcast_convert_type(p & jnp.int32(HI_MASK), jnp.float32)


def _encode_body(x_ref, encW_ref, encb_ref, w1a_ref, b1_ref, w1b_ref,
                 h_ref, a_ref, b_ref):
    h = jnp.tanh(_mm(x_ref[...], encW_ref[...]) + encb_ref[...])
    h_ref[...] = h
    a_ref[...] = _pack_cols(_mm(h, w1a_ref[...]) + b1_ref[...])
    b_ref[...] = _pack_cols(_mm(h, w1b_ref[...]))


def _edge_mlp_body(ga_ref, gb_ref, w2_ref, b2_ref, t2_ref):
    pa = ga_ref[...]
    pb = gb_ref[...]
    lo = _unpack_lo(pa) + _unpack_lo(pb)
    hi = _unpack_hi(pa) + _unpack_hi(pb)
    t1 = jnp.tanh(jnp.concatenate([lo, hi], axis=1))
    t2_ref[...] = jnp.tanh(_mm(t1, w2_ref[...]) + b2_ref[...])


def _gru_core(s0_ref, s1_ref, c0_ref, c1_ref, h_ref, w3_ref, b3_ref,
              wih_ref, bih_ref, whh_ref, bhh_ref):
    s = s0_ref[...] + s1_ref[...]
    c = c0_ref[...][:, 0:1] + c1_ref[...][:, 0:1]
    aggr = _mm(s / c, w3_ref[...]) + b3_ref[...]
    h = h_ref[...]
    gi = _mm(aggr, wih_ref[...]) + bih_ref[...]
    gh = _mm(h, whh_ref[...]) + bhh_ref[...]
    r = jax.nn.sigmoid(gi[:, :H] + gh[:, :H])
    z = jax.nn.sigmoid(gi[:, H:2 * H] + gh[:, H:2 * H])
    n = jnp.tanh(gi[:, 2 * H:] + r * gh[:, 2 * H:])
    return (1.0 - z) * n + z * h


def _gru_next_body(s0_ref, s1_ref, c0_ref, c1_ref, h_ref, w3_ref, b3_ref,
                   wih_ref, bih_ref, whh_ref, bhh_ref,
                   w1a_ref, b1_ref, w1b_ref, h_out_ref, a_ref, b_ref):
    hn = _gru_core(s0_ref, s1_ref, c0_ref, c1_ref, h_ref, w3_ref, b3_ref,
                   wih_ref, bih_ref, whh_ref, bhh_ref)
    h_out_ref[...] = hn
    a_ref[...] = _pack_cols(_mm(hn, w1a_ref[...]) + b1_ref[...])
    b_ref[...] = _pack_cols(_mm(hn, w1b_ref[...]))


def _gru_decode_body(s0_ref, s1_ref, c0_ref, c1_ref, h_ref, w3_ref, b3_ref,
                     wih_ref, bih_ref, whh_ref, bhh_ref,
                     dw1_ref, db1_ref, dw2_ref, db2_ref, dw3_ref, db3_ref,
                     o_ref):
    hn = _gru_core(s0_ref, s1_ref, c0_ref, c1_ref, h_ref, w3_ref, b3_ref,
                   wih_ref, bih_ref, whh_ref, bhh_ref)
    d = jnp.tanh(_mm(hn, dw1_ref[...]) + db1_ref[...])
    d = jnp.tanh(_mm(d, dw2_ref[...]) + db2_ref[...])
    o_ref[...] = _mm(d, dw3_ref[...]) + db3_ref[...]


def _node_spec():
    return pl.BlockSpec((BN, 128), lambda i: (i, 0))


def _pack_spec():
    return pl.BlockSpec((BN, 64), lambda i: (i, 0))


def _packed_node_out():
    return [jax.ShapeDtypeStruct((NPAD, 128), jnp.float32),
            jax.ShapeDtypeStruct((NPAD, 64), jnp.int32),
            jax.ShapeDtypeStruct((NPAD, 64), jnp.int32)]


def _full_spec(shape):
    nd = len(shape)
    return pl.BlockSpec(shape, lambda i: (0,) * nd)


def _encode_call(x, encW, encb, w1a, b1, w1b):
    return pl.pallas_call(
        _encode_body,
        grid=(NPAD // BN,),
        in_specs=[_node_spec(), _full_spec((128, 128)), _full_spec((1, 128)),
                  _full_spec((128, 128)), _full_spec((1, 128)),
                  _full_spec((128, 128))],
        out_specs=[_node_spec(), _pack_spec(), _pack_spec()],
        out_shape=_packed_node_out(),
    )(x, encW, encb, w1a, b1, w1b)


def _edge_mlp_call(ga, gb, w2, b2):
    espec = pl.BlockSpec((BE, 128), lambda i: (i, 0))
    pspec = pl.BlockSpec((BE, 64), lambda i: (i, 0))
    return pl.pallas_call(
        _edge_mlp_body,
        grid=(EPAD // BE,),
        in_specs=[pspec, pspec, _full_spec((128, 128)), _full_spec((1, 128))],
        out_specs=espec,
        out_shape=jax.ShapeDtypeStruct((EPAD, 128), jnp.float32),
    )(ga, gb, w2, b2)


def _gru_next_call(s0, s1, c0, c1, h, w3, b3, wih, bih, whh, bhh, w1a, b1, w1b):
    cspec = pl.BlockSpec((BN, 16), lambda i: (i, 0))
    return pl.pallas_call(
        _gru_next_body,
        grid=(NPAD // BN,),
        in_specs=[_node_spec(), _node_spec(), cspec, cspec, _node_spec(),
                  _full_spec((128, 128)), _full_spec((1, 128)),
                  _full_spec((128, 384)), _full_spec((1, 384)),
                  _full_spec((128, 384)), _full_spec((1, 384)),
                  _full_spec((128, 128)), _full_spec((1, 128)),
                  _full_spec((128, 128))],
        out_specs=[_node_spec(), _pack_spec(), _pack_spec()],
        out_shape=_packed_node_out(),
    )(s0, s1, c0, c1, h, w3, b3, wih, bih, whh, bhh, w1a, b1, w1b)


def _gru_decode_call(s0, s1, c0, c1, h, w3, b3, wih, bih, whh, bhh,
                     dw1, db1, dw2, db2, dw3, db3):
    cspec = pl.BlockSpec((BN, 16), lambda i: (i, 0))
    return pl.pallas_call(
        _gru_decode_body,
        grid=(NPAD // BN,),
        in_specs=[_node_spec(), _node_spec(), cspec, cspec, _node_spec(),
                  _full_spec((128, 128)), _full_spec((1, 128)),
                  _full_spec((128, 384)), _full_spec((1, 384)),
                  _full_spec((128, 384)), _full_spec((1, 384)),
                  _full_spec((128, 128)), _full_spec((1, 128)),
                  _full_spec((128, 128)), _full_spec((1, 128)),
                  _full_spec((128, 128)), _full_spec((1, 128))],
        out_specs=_node_spec(),
        out_shape=jax.ShapeDtypeStruct((NPAD, 128), jnp.float32),
    )(s0, s1, c0, c1, h, w3, b3, wih, bih, whh, bhh,
      dw1, db1, dw2, db2, dw3, db3)


def _sc_gather(a_tab, b_tab, dst_r, src_r):
    @functools.partial(
        pl.kernel,
        out_type=[jax.ShapeDtypeStruct((EPAD, 64), jnp.int32)] * 2,
        mesh=_sc_mesh(),
        compiler_params=pltpu.CompilerParams(use_tc_tiling_on_sc=False))
    def k(a_hbm, b_hbm, di_hbm, si_hbm, ga_hbm, gb_hbm):
        def body(di_v, si_v, ga_v, gb_v):
            pltpu.sync_copy(a_hbm.at[di_v.at[0]], ga_v)
            pltpu.sync_copy(b_hbm.at[si_v.at[0]], gb_v)

        pltpu.emit_pipeline(
            body,
            grid=(EPAD // W,),
            in_specs=[pl.BlockSpec((1, W), lambda i: (0, i)),
                      pl.BlockSpec((1, W), lambda i: (0, i))],
            out_specs=[pl.BlockSpec((W, 64), lambda i: (i, 0)),
                       pl.BlockSpec((W, 64), lambda i: (i, 0))],
            core_axis_name=("core", "subcore"),
            dimension_semantics=(pltpu.PARALLEL,),
        )(di_hbm, si_hbm, ga_hbm, gb_hbm)

    return k(a_tab, b_tab, dst_r, src_r)


def _sc_scatter_add(t2, dst_r, zeros_n):
    @functools.partial(
        pl.kernel,
        out_type=jax.ShapeDtypeStruct((2, NPAD, 128), jnp.float32),
        mesh=_sc_mesh(),
        scratch_types=[pltpu.VMEM_SHARED((NPAD, 128), jnp.float32)])
    def k(t2_hbm, di_hbm, z_hbm, o_hbm, s_sh):
        core = lax.axis_index("core")
        sub = lax.axis_index("subcore")
        sl = pl.ds(sub * ROWS_PER_SUB, ROWS_PER_SUB)
        pltpu.sync_copy(z_hbm.at[sl], s_sh.at[sl])
        plsc.subcore_barrier()

        def body(t2_v, di_v):
            pltpu.sync_copy(t2_v, s_sh.at[di_v.at[0]], add=True)

        pltpu.emit_pipeline(
            body,
            grid=(EPAD // W,),
            in_specs=[pl.BlockSpec((W, 128), lambda i: (i, 0)),
                      pl.BlockSpec((1, W), lambda i: (0, i))],
            core_axis_name=("core", "subcore"),
            dimension_semantics=(pltpu.PARALLEL,),
        )(t2_hbm, di_hbm)

        plsc.subcore_barrier()
        pltpu.sync_copy(s_sh.at[sl], o_hbm.at[core, sl])

    return k(t2, dst_r, zeros_n)


def _sc_count(dst_r, ones_w, zeros_c):
    @functools.partial(
        pl.kernel,
        out_type=jax.ShapeDtypeStruct((2, NPAD, 16), jnp.float32),
        mesh=_sc_mesh(),
        scratch_types=[pltpu.VMEM_SHARED((NPAD, 16), jnp.float32)])
    def k(di_hbm, ones_hbm, z_hbm, o_hbm, c_sh):
        core = lax.axis_index("core")
        sub = lax.axis_index("subcore")
        sl = pl.ds(sub * ROWS_PER_SUB, ROWS_PER_SUB)
        pltpu.sync_copy(z_hbm.at[sl], c_sh.at[sl])
        plsc.subcore_barrier()

        def body(ones_v, di_v):
            pltpu.sync_copy(ones_v, c_sh.at[di_v.at[0]], add=True)

        pltpu.emit_pipeline(
            body,
            grid=(EPAD // W,),
            in_specs=[pl.BlockSpec((W, 16), lambda i: (0, 0)),
                      pl.BlockSpec((1, W), lambda i: (0, i))],
            core_axis_name=("core", "subcore"),
            dimension_semantics=(pltpu.PARALLEL,),
        )(ones_hbm, di_hbm)

        plsc.subcore_barrier()
        pltpu.sync_copy(c_sh.at[sl], o_hbm.at[core, sl])

    return k(dst_r, ones_w, zeros_c)


def kernel(x, edge_index, batch, enc_W, enc_b, mW1, mb1, mW2, mb2, mW3, mb3,
           gWih, gWhh, gbih, gbhh, dW1, db1, dW2, db2, dW3, db3):
    del batch
    f32 = jnp.float32
    loops = jnp.arange(N, dtype=edge_index.dtype)
    src = jnp.concatenate([edge_index[0], loops])
    dst = jnp.concatenate([edge_index[1], loops])
    pad = jnp.full((EPAD - ETOT,), NPAD - 1, dtype=edge_index.dtype)
    src_r = jnp.concatenate([src, pad]).reshape(1, EPAD)
    dst_r = jnp.concatenate([dst, pad]).reshape(1, EPAD)

    x_pad = jnp.zeros((NPAD, x.shape[1]), f32).at[:N].set(x)
    zeros_n = jnp.zeros((NPAD, 128), f32)
    zeros_c = jnp.zeros((NPAD, 16), f32)
    ones_w = jnp.ones((W, 16), f32)

    encb_r = enc_b.reshape(1, 128)
    dw3_pad = jnp.zeros((128, 128), f32).at[:, 0:1].set(dW3)
    db3_pad = jnp.zeros((1, 128), f32).at[0, 0].set(db3[0])

    cnt = _sc_count(dst_r, ones_w, zeros_c)

    h, a_tab, b_tab = _encode_call(
        x_pad, enc_W, encb_r, mW1[0, :H], mb1[0].reshape(1, 128), mW1[0, H:])

    for i in range(2):
        ga, gb = _sc_gather(a_tab, b_tab, dst_r, src_r)
        t2 = _edge_mlp_call(ga, gb, mW2[i], mb2[i].reshape(1, 128))
        s_part = _sc_scatter_add(t2, dst_r, zeros_n)
        if i == 0:
            h, a_tab, b_tab = _gru_next_call(
                s_part[0], s_part[1], cnt[0], cnt[1], h,
                mW3[0], mb3[0].reshape(1, 128),
                gWih[0], gbih[0].reshape(1, 384), gWhh[0], gbhh[0].reshape(1, 384),
                mW1[1, :H], mb1[1].reshape(1, 128), mW1[1, H:])
        else:
            o = _gru_decode_call(
                s_part[0], s_part[1], cnt[0], cnt[1], h,
                mW3[1], mb3[1].reshape(1, 128),
                gWih[1], gbih[1].reshape(1, 384), gWhh[1], gbhh[1].reshape(1, 384),
                dW1, db1.reshape(1, 128), dW2, db2.reshape(1, 128),
                dw3_pad, db3_pad)

    return o[:N, 0].reshape(100, 100)


# speedup vs baseline: 1.4175x; 1.4175x over previous
import functools

import jax
import jax.numpy as jnp
from jax import lax
from jax.experimental import pallas as pl
from jax.experimental.pallas import tpu as pltpu
from jax.experimental.pallas import tpu_sc as plsc

N = 10000
NPAD = 10240
E = 160000
ETOT = 170000
EPAD = 170240
H = 128
W = 128
BN = 1280
EH = EPAD // 2
BE = 1064
NSUB = 16
ROWS_PER_SUB = NPAD // NSUB

@functools.cache
def _sc_mesh():
    return plsc.VectorSubcoreMesh(core_axis_name="core",
                                  subcore_axis_name="subcore")


def _mm(a, b):
    return jax.lax.dot_general(
        a, b, (((1,), (0,)), ((), ())),
        precision=jax.lax.Precision.HIGHEST,
        preferred_element_type=jnp.float32)


HI_MASK = -65536


def _pack_cols(v):
    lo = v[:, :64].astype(jnp.bfloat16).astype(jnp.float32)
    hi = v[:, 64:].astype(jnp.bfloat16).astype(jnp.float32)
    lo_i = jax.lax.bitcast_convert_type(lo, jnp.int32)
    hi_i = jax.lax.bitcast_convert_type(hi, jnp.int32)
    return jax.lax.shift_right_logical(lo_i, 16) | (hi_i & jnp.int32(HI_MASK))


def _unpack_lo(p):
    return jax.lax.bitcast_convert_type(jax.lax.shift_left(p, 16), jnp.float32)


def _unpack_hi(p):
    return jax.lax.bitcast_convert_type(p & jnp.int32(HI_MASK), jnp.float32)


def _encode_body(x_ref, encW_ref, encb_ref, w1a_ref, b1_ref, w1b_ref,
                 h_ref, a_ref, b_ref):
    h = jnp.tanh(_mm(x_ref[...], encW_ref[...]) + encb_ref[...])
    h_ref[...] = h
    a_ref[...] = _pack_cols(_mm(h, w1a_ref[...]) + b1_ref[...])
    b_ref[...] = _pack_cols(_mm(h, w1b_ref[...]))


def _mmbf(a, b):
    return jax.lax.dot_general(
        a, b, (((1,), (0,)), ((), ())),
        preferred_element_type=jnp.float32)


def _edge_mlp_body(ga_ref, gb_ref, w2ea_ref, w2eb_ref, w2oa_ref, w2ob_ref,
                   b2_ref, te_ref, to_ref):
    pa = ga_ref[...]
    pb = gb_ref[...]
    q = jnp.tanh(_unpack_lo(pa) + _unpack_lo(pb)).astype(jnp.bfloat16)
    r = jnp.tanh(_unpack_hi(pa) + _unpack_hi(pb)).astype(jnp.bfloat16)
    b2 = b2_ref[...]
    te = _mmbf(q, w2ea_ref[...]) + _mmbf(r, w2eb_ref[...]) + b2
    to = _mmbf(q, w2oa_ref[...]) + _mmbf(r, w2ob_ref[...]) + b2
    te_ref[...] = jnp.tanh(te)
    to_ref[...] = jnp.tanh(to)


def _gru_core(s0_ref, s1_ref, c0_ref, c1_ref, h_ref, w3_ref, b3_ref,
              wih_ref, bih_ref, whh_ref, bhh_ref):
    s = s0_ref[...] + s1_ref[...]
    c = c0_ref[...][:, 0:1] + c1_ref[...][:, 0:1]
    aggr = _mm(s / c, w3_ref[...]) + b3_ref[...]
    h = h_ref[...]
    gi = _mm(aggr, wih_ref[...]) + bih_ref[...]
    gh = _mm(h, whh_ref[...]) + bhh_ref[...]
    r = jax.nn.sigmoid(gi[:, :H] + gh[:, :H])
    z = jax.nn.sigmoid(gi[:, H:2 * H] + gh[:, H:2 * H])
    n = jnp.tanh(gi[:, 2 * H:] + r * gh[:, 2 * H:])
    return (1.0 - z) * n + z * h


def _gru_next_body(s0_ref, s1_ref, c0_ref, c1_ref, h_ref, w3_ref, b3_ref,
                   wih_ref, bih_ref, whh_ref, bhh_ref,
                   w1a_ref, b1_ref, w1b_ref, h_out_ref, a_ref, b_ref):
    hn = _gru_core(s0_ref, s1_ref, c0_ref, c1_ref, h_ref, w3_ref, b3_ref,
                   wih_ref, bih_ref, whh_ref, bhh_ref)
    h_out_ref[...] = hn
    a_ref[...] = _pack_cols(_mm(hn, w1a_ref[...]) + b1_ref[...])
    b_ref[...] = _pack_cols(_mm(hn, w1b_ref[...]))


def _gru_decode_body(s0_ref, s1_ref, c0_ref, c1_ref, h_ref, w3_ref, b3_ref,
                     wih_ref, bih_ref, whh_ref, bhh_ref,
                     dw1_ref, db1_ref, dw2_ref, db2_ref, dw3_ref, db3_ref,
                     o_ref):
    hn = _gru_core(s0_ref, s1_ref, c0_ref, c1_ref, h_ref, w3_ref, b3_ref,
                   wih_ref, bih_ref, whh_ref, bhh_ref)
    d = jnp.tanh(_mm(hn, dw1_ref[...]) + db1_ref[...])
    d = jnp.tanh(_mm(d, dw2_ref[...]) + db2_ref[...])
    o_ref[...] = _mm(d, dw3_ref[...]) + db3_ref[...]


def _node_spec():
    return pl.BlockSpec((BN, 128), lambda i: (i, 0))


def _pack_spec():
    return pl.BlockSpec((BN, 64), lambda i: (i, 0))


def _packed_node_out():
    return [jax.ShapeDtypeStruct((NPAD, 128), jnp.float32),
            jax.ShapeDtypeStruct((NPAD, 64), jnp.int32),
            jax.ShapeDtypeStruct((NPAD, 64), jnp.int32)]


def _full_spec(shape):
    nd = len(shape)
    return pl.BlockSpec(shape, lambda i: (0,) * nd)


def _encode_call(x, encW, encb, w1a, b1, w1b):
    return pl.pallas_call(
        _encode_body,
        grid=(NPAD // BN,),
        in_specs=[_node_spec(), _full_spec((128, 128)), _full_spec((1, 128)),
                  _full_spec((128, 128)), _full_spec((1, 128)),
                  _full_spec((128, 128))],
        out_specs=[_node_spec(), _pack_spec(), _pack_spec()],
        out_shape=_packed_node_out(),
    )(x, encW, encb, w1a, b1, w1b)


def _edge_mlp_call(ga, gb, w2s, b2):
    espec = pl.BlockSpec((BE, 128), lambda i: (i, 0))
    return pl.pallas_call(
        _edge_mlp_body,
        grid=(EH // BE,),
        in_specs=[espec, espec,
                  _full_spec((128, 128)), _full_spec((128, 128)),
                  _full_spec((128, 128)), _full_spec((128, 128)),
                  _full_spec((1, 128))],
        out_specs=[espec, espec],
        out_shape=[jax.ShapeDtypeStruct((EH, 128), jnp.float32)] * 2,
    )(ga, gb, w2s[0], w2s[1], w2s[2], w2s[3], b2)


def _gru_next_call(s0, s1, c0, c1, h, w3, b3, wih, bih, whh, bhh, w1a, b1, w1b):
    cspec = pl.BlockSpec((BN, 16), lambda i: (i, 0))
    return pl.pallas_call(
        _gru_next_body,
        grid=(NPAD // BN,),
        in_specs=[_node_spec(), _node_spec(), cspec, cspec, _node_spec(),
                  _full_spec((128, 128)), _full_spec((1, 128)),
                  _full_spec((128, 384)), _full_spec((1, 384)),
                  _full_spec((128, 384)), _full_spec((1, 384)),
                  _full_spec((128, 128)), _full_spec((1, 128)),
                  _full_spec((128, 128))],
        out_specs=[_node_spec(), _pack_spec(), _pack_spec()],
        out_shape=_packed_node_out(),
    )(s0, s1, c0, c1, h, w3, b3, wih, bih, whh, bhh, w1a, b1, w1b)


def _gru_decode_call(s0, s1, c0, c1, h, w3, b3, wih, bih, whh, bhh,
                     dw1, db1, dw2, db2, dw3, db3):
    cspec = pl.BlockSpec((BN, 16), lambda i: (i, 0))
    return pl.pallas_call(
        _gru_decode_body,
        grid=(NPAD // BN,),
        in_specs=[_node_spec(), _node_spec(), cspec, cspec, _node_spec(),
                  _full_spec((128, 128)), _full_spec((1, 128)),
                  _full_spec((128, 384)), _full_spec((1, 384)),
                  _full_spec((128, 384)), _full_spec((1, 384)),
                  _full_spec((128, 128)), _full_spec((1, 128)),
                  _full_spec((128, 128)), _full_spec((1, 128)),
                  _full_spec((128, 128)), _full_spec((1, 128))],
        out_specs=_node_spec(),
        out_shape=jax.ShapeDtypeStruct((NPAD, 128), jnp.float32),
    )(s0, s1, c0, c1, h, w3, b3, wih, bih, whh, bhh,
      dw1, db1, dw2, db2, dw3, db3)


def _sc_gather(a_tab, b_tab, dst_r, src_r):
    @functools.partial(
        pl.kernel,
        out_type=[jax.ShapeDtypeStruct((EPAD, 64), jnp.int32)] * 2,
        mesh=_sc_mesh(),
        compiler_params=pltpu.CompilerParams(use_tc_tiling_on_sc=False))
    def k(a_hbm, b_hbm, di_hbm, si_hbm, ga_hbm, gb_hbm):
        def body(di_v, si_v, ga_v, gb_v):
            pltpu.sync_copy(a_hbm.at[di_v.at[0]], ga_v)
            pltpu.sync_copy(b_hbm.at[si_v.at[0]], gb_v)

        pltpu.emit_pipeline(
            body,
            grid=(EPAD // W,),
            in_specs=[pl.BlockSpec((1, W), lambda i: (0, i)),
                      pl.BlockSpec((1, W), lambda i: (0, i))],
            out_specs=[pl.BlockSpec((W, 64), lambda i: (i, 0)),
                       pl.BlockSpec((W, 64), lambda i: (i, 0))],
            core_axis_name=("core", "subcore"),
            dimension_semantics=(pltpu.PARALLEL,),
        )(di_hbm, si_hbm, ga_hbm, gb_hbm)

    return k(a_tab, b_tab, dst_r, src_r)


def _sc_scatter_add(te, to, de_r, do_r, zeros_n):
    @functools.partial(
        pl.kernel,
        out_type=jax.ShapeDtypeStruct((2, NPAD, 128), jnp.float32),
        mesh=_sc_mesh(),
        scratch_types=[pltpu.VMEM_SHARED((NPAD, 128), jnp.float32)])
    def k(te_hbm, to_hbm, de_hbm, do_hbm, z_hbm, o_hbm, s_sh):
        core = lax.axis_index("core")
        sub = lax.axis_index("subcore")
        sl = pl.ds(sub * ROWS_PER_SUB, ROWS_PER_SUB)
        pltpu.sync_copy(z_hbm.at[sl], s_sh.at[sl])
        plsc.subcore_barrier()

        def body(t_v, d_v):
            pltpu.sync_copy(t_v, s_sh.at[d_v.at[0]], add=True)

        for t_hbm, d_hbm in ((te_hbm, de_hbm), (to_hbm, do_hbm)):
            pltpu.emit_pipeline(
                body,
                grid=(EH // W,),
                in_specs=[pl.BlockSpec((W, 128), lambda i: (i, 0)),
                          pl.BlockSpec((1, W), lambda i: (0, i))],
                core_axis_name=("core", "subcore"),
                dimension_semantics=(pltpu.PARALLEL,),
            )(t_hbm, d_hbm)

        plsc.subcore_barrier()
        pltpu.sync_copy(s_sh.at[sl], o_hbm.at[core, sl])

    return k(te, to, de_r, do_r, zeros_n)


def _sc_count(dst_r, ones_w, zeros_c):
    @functools.partial(
        pl.kernel,
        out_type=jax.ShapeDtypeStruct((2, NPAD, 16), jnp.float32),
        mesh=_sc_mesh(),
        scratch_types=[pltpu.VMEM_SHARED((NPAD, 16), jnp.float32)])
    def k(di_hbm, ones_hbm, z_hbm, o_hbm, c_sh):
        core = lax.axis_index("core")
        sub = lax.axis_index("subcore")
        sl = pl.ds(sub * ROWS_PER_SUB, ROWS_PER_SUB)
        pltpu.sync_copy(z_hbm.at[sl], c_sh.at[sl])
        plsc.subcore_barrier()

        def body(ones_v, di_v):
            pltpu.sync_copy(ones_v, c_sh.at[di_v.at[0]], add=True)

        pltpu.emit_pipeline(
            body,
            grid=(EPAD // W,),
            in_specs=[pl.BlockSpec((W, 16), lambda i: (0, 0)),
                      pl.BlockSpec((1, W), lambda i: (0, i))],
            core_axis_name=("core", "subcore"),
            dimension_semantics=(pltpu.PARALLEL,),
        )(ones_hbm, di_hbm)

        plsc.subcore_barrier()
        pltpu.sync_copy(c_sh.at[sl], o_hbm.at[core, sl])

    return k(dst_r, ones_w, zeros_c)


def kernel(x, edge_index, batch, enc_W, enc_b, mW1, mb1, mW2, mb2, mW3, mb3,
           gWih, gWhh, gbih, gbhh, dW1, db1, dW2, db2, dW3, db3):
    del batch
    f32 = jnp.float32
    loops = jnp.arange(N, dtype=edge_index.dtype)
    src = jnp.concatenate([edge_index[0], loops])
    dst = jnp.concatenate([edge_index[1], loops])
    pad = jnp.full((EPAD - ETOT,), NPAD - 1, dtype=edge_index.dtype)
    src_full = jnp.concatenate([src, pad])
    dst_full = jnp.concatenate([dst, pad])
    src_r = src_full.reshape(1, EPAD)
    dst_r = dst_full.reshape(1, EPAD)
    de_r = dst_full[0::2].reshape(1, EH)
    do_r = dst_full[1::2].reshape(1, EH)

    x_pad = jnp.zeros((NPAD, x.shape[1]), f32).at[:N].set(x)
    zeros_n = jnp.zeros((NPAD, 128), f32)
    zeros_c = jnp.zeros((NPAD, 16), f32)
    ones_w = jnp.ones((W, 16), f32)

    encb_r = enc_b.reshape(1, 128)
    dw3_pad = jnp.zeros((128, 128), f32).at[:, 0:1].set(dW3)
    db3_pad = jnp.zeros((1, 128), f32).at[0, 0].set(db3[0])

    cnt = _sc_count(dst_r, ones_w, zeros_c)

    h, a_tab, b_tab = _encode_call(
        x_pad, enc_W, encb_r, mW1[0, :H], mb1[0].reshape(1, 128), mW1[0, H:])

    z64 = jnp.zeros((64, 128), f32)
    w2s = []
    for i in range(2):
        wtop, wbot = mW2[i][:64], mW2[i][64:]
        w2s.append([jnp.concatenate([wtop, z64]).astype(jnp.bfloat16),
                    jnp.concatenate([wbot, z64]).astype(jnp.bfloat16),
                    jnp.concatenate([z64, wtop]).astype(jnp.bfloat16),
                    jnp.concatenate([z64, wbot]).astype(jnp.bfloat16)])

    for i in range(2):
        ga, gb = _sc_gather(a_tab, b_tab, dst_r, src_r)
        te, to = _edge_mlp_call(ga.reshape(EH, 128), gb.reshape(EH, 128),
                                w2s[i], mb2[i].reshape(1, 128))
        s_part = _sc_scatter_add(te, to, de_r, do_r, zeros_n)
        if i == 0:
            h, a_tab, b_tab = _gru_next_call(
                s_part[0], s_part[1], cnt[0], cnt[1], h,
                mW3[0], mb3[0].reshape(1, 128),
                gWih[0], gbih[0].reshape(1, 384), gWhh[0], gbhh[0].reshape(1, 384),
                mW1[1, :H], mb1[1].reshape(1, 128), mW1[1, H:])
        else:
            o = _gru_decode_call(
                s_part[0], s_part[1], cnt[0], cnt[1], h,
                mW3[1], mb3[1].reshape(1, 128),
                gWih[1], gbih[1].reshape(1, 384), gWhh[1], gbhh[1].reshape(1, 384),
                dW1, db1.reshape(1, 128), dW2, db2.reshape(1, 128),
                dw3_pad, db3_pad)

    return o[:N, 0].reshape(100, 100)


# speedup vs baseline: 1.4668x; 1.0348x over previous
import functools

import jax
import jax.numpy as jnp
from jax import lax
from jax.experimental import pallas as pl
from jax.experimental.pallas import tpu as pltpu
from jax.experimental.pallas import tpu_sc as plsc

N = 10000
NPAD = 10240
E = 160000
ETOT = 170000
EPAD = 170496
H = 128
W = 128
BN = 1280
EH = EPAD // 2
CK = 2
EC = EPAD // CK
EHC = EH // CK
BE = 888
NSUB = 16
ROWS_PER_SUB = NPAD // NSUB

@functools.cache
def _sc_mesh():
    return plsc.VectorSubcoreMesh(core_axis_name="core",
                                  subcore_axis_name="subcore")


def _mm(a, b):
    return jax.lax.dot_general(
        a, b, (((1,), (0,)), ((), ())),
        precision=jax.lax.Precision.HIGHEST,
        preferred_element_type=jnp.float32)


HI_MASK = -65536


def _pack_cols(v):
    lo = v[:, :64].astype(jnp.bfloat16).astype(jnp.float32)
    hi = v[:, 64:].astype(jnp.bfloat16).astype(jnp.float32)
    lo_i = jax.lax.bitcast_convert_type(lo, jnp.int32)
    hi_i = jax.lax.bitcast_convert_type(hi, jnp.int32)
    return jax.lax.shift_right_logical(lo_i, 16) | (hi_i & jnp.int32(HI_MASK))


def _unpack_lo(p):
    return jax.lax.bitcast_convert_type(jax.lax.shift_left(p, 16), jnp.float32)


def _unpack_hi(p):
    return jax.lax.bitcast_convert_type(p & jnp.int32(HI_MASK), jnp.float32)


def _encode_body(x_ref, encW_ref, encb_ref, w1a_ref, b1_ref, w1b_ref,
                 h_ref, a_ref, b_ref):
    h = jnp.tanh(_mm(x_ref[...], encW_ref[...]) + encb_ref[...])
    h_ref[...] = h
    a_ref[...] = _pack_cols(_mm(h, w1a_ref[...]) + b1_ref[...])
    b_ref[...] = _pack_cols(_mm(h, w1b_ref[...]))


def _mmbf(a, b):
    return jax.lax.dot_general(
        a, b, (((1,), (0,)), ((), ())),
        preferred_element_type=jnp.float32)


def _edge_mlp_body(ga_ref, gb_ref, w2ea_ref, w2eb_ref, w2oa_ref, w2ob_ref,
                   b2_ref, te_ref, to_ref):
    pa = ga_ref[...]
    pb = gb_ref[...]
    q = jnp.tanh(_unpack_lo(pa) + _unpack_lo(pb)).astype(jnp.bfloat16)
    r = jnp.tanh(_unpack_hi(pa) + _unpack_hi(pb)).astype(jnp.bfloat16)
    b2 = b2_ref[...]
    te = _mmbf(q, w2ea_ref[...]) + _mmbf(r, w2eb_ref[...]) + b2
    to = _mmbf(q, w2oa_ref[...]) + _mmbf(r, w2ob_ref[...]) + b2
    te_ref[...] = jnp.tanh(te)
    to_ref[...] = jnp.tanh(to)


def _gru_core(s0_ref, s1_ref, s2_ref, s3_ref, c0_ref, c1_ref, h_ref,
              w3_ref, b3_ref, wih_ref, bih_ref, whh_ref, bhh_ref):
    s = (s0_ref[...] + s1_ref[...]) + (s2_ref[...] + s3_ref[...])
    c = c0_ref[...][:, 0:1] + c1_ref[...][:, 0:1]
    aggr = _mm(s / c, w3_ref[...]) + b3_ref[...]
    h = h_ref[...]
    gi = _mm(aggr, wih_ref[...]) + bih_ref[...]
    gh = _mm(h, whh_ref[...]) + bhh_ref[...]
    r = jax.nn.sigmoid(gi[:, :H] + gh[:, :H])
    z = jax.nn.sigmoid(gi[:, H:2 * H] + gh[:, H:2 * H])
    n = jnp.tanh(gi[:, 2 * H:] + r * gh[:, 2 * H:])
    return (1.0 - z) * n + z * h


def _gru_next_body(s0_ref, s1_ref, s2_ref, s3_ref, c0_ref, c1_ref, h_ref,
                   w3_ref, b3_ref, wih_ref, bih_ref, whh_ref, bhh_ref,
                   w1a_ref, b1_ref, w1b_ref, h_out_ref, a_ref, b_ref):
    hn = _gru_core(s0_ref, s1_ref, s2_ref, s3_ref, c0_ref, c1_ref, h_ref,
                   w3_ref, b3_ref, wih_ref, bih_ref, whh_ref, bhh_ref)
    h_out_ref[...] = hn
    a_ref[...] = _pack_cols(_mm(hn, w1a_ref[...]) + b1_ref[...])
    b_ref[...] = _pack_cols(_mm(hn, w1b_ref[...]))


def _gru_decode_body(s0_ref, s1_ref, s2_ref, s3_ref, c0_ref, c1_ref, h_ref,
                     w3_ref, b3_ref, wih_ref, bih_ref, whh_ref, bhh_ref,
                     dw1_ref, db1_ref, dw2_ref, db2_ref, dw3_ref, db3_ref,
                     o_ref):
    hn = _gru_core(s0_ref, s1_ref, s2_ref, s3_ref, c0_ref, c1_ref, h_ref,
                   w3_ref, b3_ref, wih_ref, bih_ref, whh_ref, bhh_ref)
    d = jnp.tanh(_mm(hn, dw1_ref[...]) + db1_ref[...])
    d = jnp.tanh(_mm(d, dw2_ref[...]) + db2_ref[...])
    o_ref[...] = _mm(d, dw3_ref[...]) + db3_ref[...]


def _node_spec():
    return pl.BlockSpec((BN, 128), lambda i: (i, 0))


def _pack_spec():
    return pl.BlockSpec((BN, 64), lambda i: (i, 0))


def _packed_node_out():
    return [jax.ShapeDtypeStruct((NPAD, 128), jnp.float32),
            jax.ShapeDtypeStruct((NPAD, 64), jnp.int32),
            jax.ShapeDtypeStruct((NPAD, 64), jnp.int32)]


def _full_spec(shape):
    nd = len(shape)
    return pl.BlockSpec(shape, lambda i: (0,) * nd)


def _encode_call(x, encW, encb, w1a, b1, w1b):
    return pl.pallas_call(
        _encode_body,
        grid=(NPAD // BN,),
        in_specs=[_node_spec(), _full_spec((128, 128)), _full_spec((1, 128)),
                  _full_spec((128, 128)), _full_spec((1, 128)),
                  _full_spec((128, 128))],
        out_specs=[_node_spec(), _pack_spec(), _pack_spec()],
        out_shape=_packed_node_out(),
    )(x, encW, encb, w1a, b1, w1b)


def _edge_mlp_call(ga, gb, w2s, b2):
    espec = pl.BlockSpec((BE, 128), lambda i: (i, 0))
    return pl.pallas_call(
        _edge_mlp_body,
        grid=(EHC // BE,),
        in_specs=[espec, espec,
                  _full_spec((128, 128)), _full_spec((128, 128)),
                  _full_spec((128, 128)), _full_spec((128, 128)),
                  _full_spec((1, 128))],
        out_specs=[espec, espec],
        out_shape=[jax.ShapeDtypeStruct((EHC, 128), jnp.float32)] * 2,
    )(ga, gb, w2s[0], w2s[1], w2s[2], w2s[3], b2)


def _gru_next_call(s0, s1, s2, s3, c0, c1, h, w3, b3, wih, bih, whh, bhh,
                   w1a, b1, w1b):
    cspec = pl.BlockSpec((BN, 16), lambda i: (i, 0))
    return pl.pallas_call(
        _gru_next_body,
        grid=(NPAD // BN,),
        in_specs=[_node_spec(), _node_spec(), _node_spec(), _node_spec(),
                  cspec, cspec, _node_spec(),
                  _full_spec((128, 128)), _full_spec((1, 128)),
                  _full_spec((128, 384)), _full_spec((1, 384)),
                  _full_spec((128, 384)), _full_spec((1, 384)),
                  _full_spec((128, 128)), _full_spec((1, 128)),
                  _full_spec((128, 128))],
        out_specs=[_node_spec(), _pack_spec(), _pack_spec()],
        out_shape=_packed_node_out(),
    )(s0, s1, s2, s3, c0, c1, h, w3, b3, wih, bih, whh, bhh, w1a, b1, w1b)


def _gru_decode_call(s0, s1, s2, s3, c0, c1, h, w3, b3, wih, bih, whh, bhh,
                     dw1, db1, dw2, db2, dw3, db3):
    cspec = pl.BlockSpec((BN, 16), lambda i: (i, 0))
    return pl.pallas_call(
        _gru_decode_body,
        grid=(NPAD // BN,),
        in_specs=[_node_spec(), _node_spec(), _node_spec(), _node_spec(),
                  cspec, cspec, _node_spec(),
                  _full_spec((128, 128)), _full_spec((1, 128)),
                  _full_spec((128, 384)), _full_spec((1, 384)),
                  _full_spec((128, 384)), _full_spec((1, 384)),
                  _full_spec((128, 128)), _full_spec((1, 128)),
                  _full_spec((128, 128)), _full_spec((1, 128)),
                  _full_spec((128, 128)), _full_spec((1, 128))],
        out_specs=_node_spec(),
        out_shape=jax.ShapeDtypeStruct((NPAD, 128), jnp.float32),
    )(s0, s1, s2, s3, c0, c1, h, w3, b3, wih, bih, whh, bhh,
      dw1, db1, dw2, db2, dw3, db3)


def _sc_gather(a_tab, b_tab, dst_r, src_r):
    @functools.partial(
        pl.kernel,
        out_type=[jax.ShapeDtypeStruct((EC, 64), jnp.int32)] * 2,
        mesh=_sc_mesh(),
        compiler_params=pltpu.CompilerParams(use_tc_tiling_on_sc=False))
    def k(a_hbm, b_hbm, di_hbm, si_hbm, ga_hbm, gb_hbm):
        def body(di_v, si_v, ga_v, gb_v):
            pltpu.sync_copy(a_hbm.at[di_v.at[0]], ga_v)
            pltpu.sync_copy(b_hbm.at[si_v.at[0]], gb_v)

        pltpu.emit_pipeline(
            body,
            grid=(EC // W,),
            in_specs=[pl.BlockSpec((1, W), lambda i: (0, i)),
                      pl.BlockSpec((1, W), lambda i: (0, i))],
            out_specs=[pl.BlockSpec((W, 64), lambda i: (i, 0)),
                       pl.BlockSpec((W, 64), lambda i: (i, 0))],
            core_axis_name=("core", "subcore"),
            dimension_semantics=(pltpu.PARALLEL,),
        )(di_hbm, si_hbm, ga_hbm, gb_hbm)

    return k(a_tab, b_tab, dst_r, src_r)


def _sc_scatter_add(te, to, de_r, do_r, zeros_n):
    @functools.partial(
        pl.kernel,
        out_type=jax.ShapeDtypeStruct((2, NPAD, 128), jnp.float32),
        mesh=_sc_mesh(),
        scratch_types=[pltpu.VMEM_SHARED((NPAD, 128), jnp.float32)])
    def k(te_hbm, to_hbm, de_hbm, do_hbm, z_hbm, o_hbm, s_sh):
        core = lax.axis_index("core")
        sub = lax.axis_index("subcore")
        sl = pl.ds(sub * ROWS_PER_SUB, ROWS_PER_SUB)
        pltpu.sync_copy(z_hbm.at[sl], s_sh.at[sl])
        plsc.subcore_barrier()

        def body(t_v, d_v):
            pltpu.sync_copy(t_v, s_sh.at[d_v.at[0]], add=True)

        for t_hbm, d_hbm in ((te_hbm, de_hbm), (to_hbm, do_hbm)):
            pltpu.emit_pipeline(
                body,
                grid=(EHC // W,),
                in_specs=[pl.BlockSpec((W, 128), lambda i: (i, 0)),
                          pl.BlockSpec((1, W), lambda i: (0, i))],
                core_axis_name=("core", "subcore"),
                dimension_semantics=(pltpu.PARALLEL,),
            )(t_hbm, d_hbm)

        plsc.subcore_barrier()
        pltpu.sync_copy(s_sh.at[sl], o_hbm.at[core, sl])

    return k(te, to, de_r, do_r, zeros_n)


def _sc_count(dst_r, ones_w, zeros_c):
    @functools.partial(
        pl.kernel,
        out_type=jax.ShapeDtypeStruct((2, NPAD, 16), jnp.float32),
        mesh=_sc_mesh(),
        scratch_types=[pltpu.VMEM_SHARED((NPAD, 16), jnp.float32)])
    def k(di_hbm, ones_hbm, z_hbm, o_hbm, c_sh):
        core = lax.axis_index("core")
        sub = lax.axis_index("subcore")
        sl = pl.ds(sub * ROWS_PER_SUB, ROWS_PER_SUB)
        pltpu.sync_copy(z_hbm.at[sl], c_sh.at[sl])
        plsc.subcore_barrier()

        def body(ones_v, di_v):
            pltpu.sync_copy(ones_v, c_sh.at[di_v.at[0]], add=True)

        pltpu.emit_pipeline(
            body,
            grid=(EPAD // W,),
            in_specs=[pl.BlockSpec((W, 16), lambda i: (0, 0)),
                      pl.BlockSpec((1, W), lambda i: (0, i))],
            core_axis_name=("core", "subcore"),
            dimension_semantics=(pltpu.PARALLEL,),
        )(ones_hbm, di_hbm)

        plsc.subcore_barrier()
        pltpu.sync_copy(c_sh.at[sl], o_hbm.at[core, sl])

    return k(dst_r, ones_w, zeros_c)


def kernel(x, edge_index, batch, enc_W, enc_b, mW1, mb1, mW2, mb2, mW3, mb3,
           gWih, gWhh, gbih, gbhh, dW1, db1, dW2, db2, dW3, db3):
    del batch
    f32 = jnp.float32
    loops = jnp.arange(N, dtype=edge_index.dtype)
    src = jnp.concatenate([edge_index[0], loops])
    dst = jnp.concatenate([edge_index[1], loops])
    pad = jnp.full((EPAD - ETOT,), NPAD - 1, dtype=edge_index.dtype)
    src_full = jnp.concatenate([src, pad])
    dst_full = jnp.concatenate([dst, pad])
    src_r = src_full.reshape(1, EPAD)
    dst_r = dst_full.reshape(1, EPAD)
    de_r = dst_full[0::2].reshape(1, EH)
    do_r = dst_full[1::2].reshape(1, EH)

    x_pad = jnp.zeros((NPAD, x.shape[1]), f32).at[:N].set(x)
    zeros_n = jnp.zeros((NPAD, 128), f32)
    zeros_c = jnp.zeros((NPAD, 16), f32)
    ones_w = jnp.ones((W, 16), f32)

    encb_r = enc_b.reshape(1, 128)
    dw3_pad = jnp.zeros((128, 128), f32).at[:, 0:1].set(dW3)
    db3_pad = jnp.zeros((1, 128), f32).at[0, 0].set(db3[0])

    cnt = _sc_count(dst_r, ones_w, zeros_c)

    h, a_tab, b_tab = _encode_call(
        x_pad, enc_W, encb_r, mW1[0, :H], mb1[0].reshape(1, 128), mW1[0, H:])

    z64 = jnp.zeros((64, 128), f32)
    w2s = []
    for i in range(2):
        wtop, wbot = mW2[i][:64], mW2[i][64:]
        w2s.append([jnp.concatenate([wtop, z64]).astype(jnp.bfloat16),
                    jnp.concatenate([wbot, z64]).astype(jnp.bfloat16),
                    jnp.concatenate([z64, wtop]).astype(jnp.bfloat16),
                    jnp.concatenate([z64, wbot]).astype(jnp.bfloat16)])

    for i in range(2):
        parts = []
        for c in range(CK):
            esl = slice(c * EC, (c + 1) * EC)
            hsl = slice(c * EHC, (c + 1) * EHC)
            ga, gb = _sc_gather(a_tab, b_tab, dst_r[:, esl], src_r[:, esl])
            te, to = _edge_mlp_call(ga.reshape(EHC, 128), gb.reshape(EHC, 128),
                                    w2s[i], mb2[i].reshape(1, 128))
            parts.append(_sc_scatter_add(te, to, de_r[:, hsl], do_r[:, hsl],
                                         zeros_n))
        s00, s01 = parts[0][0], parts[0][1]
        s10, s11 = parts[1][0], parts[1][1]
        if i == 0:
            h, a_tab, b_tab = _gru_next_call(
                s00, s01, s10, s11, cnt[0], cnt[1], h,
                mW3[0], mb3[0].reshape(1, 128),
                gWih[0], gbih[0].reshape(1, 384), gWhh[0], gbhh[0].reshape(1, 384),
                mW1[1, :H], mb1[1].reshape(1, 128), mW1[1, H:])
        else:
            o = _gru_decode_call(
                s00, s01, s10, s11, cnt[0], cnt[1], h,
                mW3[1], mb3[1].reshape(1, 128),
                gWih[1], gbih[1].reshape(1, 384), gWhh[1], gbhh[1].reshape(1, 384),
                dW1, db1.reshape(1, 128), dW2, db2.reshape(1, 128),
                dw3_pad, db3_pad)

    return o[:N, 0].reshape(100, 100)


# speedup vs baseline: 1.6320x; 1.1126x over previous
import functools

import jax
import jax.numpy as jnp
from jax import lax
from jax.experimental import pallas as pl
from jax.experimental.pallas import tpu as pltpu
from jax.experimental.pallas import tpu_sc as plsc

N = 10000
NPAD = 10240
E = 160000
ETOT = 170000
EPAD = 170496
H = 128
W = 128
BN = 1280
EH = EPAD // 2
CK = 2
EC = EPAD // CK
EHC = EH // CK
BE = 888
NSUB = 16
ROWS_PER_SUB = NPAD // NSUB

@functools.cache
def _sc_mesh():
    return plsc.VectorSubcoreMesh(core_axis_name="core",
                                  subcore_axis_name="subcore")


def _mm(a, b):
    return jax.lax.dot_general(
        a.astype(jnp.bfloat16), b.astype(jnp.bfloat16), (((1,), (0,)), ((), ())),
        preferred_element_type=jnp.float32)


def _mmx(a, b):
    return jax.lax.dot_general(
        a, b, (((1,), (0,)), ((), ())),
        precision=jax.lax.Precision.HIGHEST,
        preferred_element_type=jnp.float32)


HI_MASK = -65536


def _pack_cols(v):
    lo = v[:, :64].astype(jnp.bfloat16).astype(jnp.float32)
    hi = v[:, 64:].astype(jnp.bfloat16).astype(jnp.float32)
    lo_i = jax.lax.bitcast_convert_type(lo, jnp.int32)
    hi_i = jax.lax.bitcast_convert_type(hi, jnp.int32)
    return jax.lax.shift_right_logical(lo_i, 16) | (hi_i & jnp.int32(HI_MASK))


def _unpack_lo(p):
    return jax.lax.bitcast_convert_type(jax.lax.shift_left(p, 16), jnp.float32)


def _unpack_hi(p):
    return jax.lax.bitcast_convert_type(p & jnp.int32(HI_MASK), jnp.float32)


def _encode_body(x_ref, encW_ref, encb_ref, w1a_ref, b1_ref, w1b_ref,
                 h_ref, a_ref, b_ref):
    h = jnp.tanh(_mm(x_ref[...], encW_ref[...]) + encb_ref[...])
    h_ref[...] = h
    a_ref[...] = _pack_cols(_mm(h, w1a_ref[...]) + b1_ref[...])
    b_ref[...] = _pack_cols(_mm(h, w1b_ref[...]))


def _mmbf(a, b):
    return jax.lax.dot_general(
        a, b, (((1,), (0,)), ((), ())),
        preferred_element_type=jnp.float32)


def _edge_mlp_body(ga_ref, gb_ref, w2ea_ref, w2eb_ref, w2oa_ref, w2ob_ref,
                   b2_ref, te_ref, to_ref):
    pa = ga_ref[...]
    pb = gb_ref[...]
    q = jnp.tanh(_unpack_lo(pa) + _unpack_lo(pb)).astype(jnp.bfloat16)
    r = jnp.tanh(_unpack_hi(pa) + _unpack_hi(pb)).astype(jnp.bfloat16)
    b2 = b2_ref[...]
    te = _mmbf(q, w2ea_ref[...]) + _mmbf(r, w2eb_ref[...]) + b2
    to = _mmbf(q, w2oa_ref[...]) + _mmbf(r, w2ob_ref[...]) + b2
    te_ref[...] = jnp.tanh(te)
    to_ref[...] = jnp.tanh(to)


def _gru_core(s0_ref, s1_ref, s2_ref, s3_ref, c0_ref, c1_ref, c2_ref, c3_ref,
              h_ref, w3_ref, b3_ref, wih_ref, bih_ref, whh_ref, bhh_ref):
    s = (s0_ref[...] + s1_ref[...]) + (s2_ref[...] + s3_ref[...])
    c = ((c0_ref[...][:, 0:1] + c1_ref[...][:, 0:1])
         + (c2_ref[...][:, 0:1] + c3_ref[...][:, 0:1]))
    aggr = _mmx(s / c, w3_ref[...]) + b3_ref[...]
    h = h_ref[...]
    gi = _mm(aggr, wih_ref[...]) + bih_ref[...]
    gh = _mm(h, whh_ref[...]) + bhh_ref[...]
    r = jax.nn.sigmoid(gi[:, :H] + gh[:, :H])
    z = jax.nn.sigmoid(gi[:, H:2 * H] + gh[:, H:2 * H])
    n = jnp.tanh(gi[:, 2 * H:] + r * gh[:, 2 * H:])
    return (1.0 - z) * n + z * h


def _gru_next_body(s0_ref, s1_ref, s2_ref, s3_ref, c0_ref, c1_ref, c2_ref,
                   c3_ref, h_ref, w3_ref, b3_ref, wih_ref, bih_ref, whh_ref,
                   bhh_ref, w1a_ref, b1_ref, w1b_ref, h_out_ref, a_ref, b_ref):
    hn = _gru_core(s0_ref, s1_ref, s2_ref, s3_ref, c0_ref, c1_ref, c2_ref,
                   c3_ref, h_ref, w3_ref, b3_ref, wih_ref, bih_ref, whh_ref,
                   bhh_ref)
    h_out_ref[...] = hn
    a_ref[...] = _pack_cols(_mm(hn, w1a_ref[...]) + b1_ref[...])
    b_ref[...] = _pack_cols(_mm(hn, w1b_ref[...]))


def _gru_decode_body(s0_ref, s1_ref, s2_ref, s3_ref, c0_ref, c1_ref, c2_ref,
                     c3_ref, h_ref, w3_ref, b3_ref, wih_ref, bih_ref, whh_ref,
                     bhh_ref, dw1_ref, db1_ref, dw2_ref, db2_ref, dw3_ref,
                     db3_ref, o_ref):
    hn = _gru_core(s0_ref, s1_ref, s2_ref, s3_ref, c0_ref, c1_ref, c2_ref,
                   c3_ref, h_ref, w3_ref, b3_ref, wih_ref, bih_ref, whh_ref,
                   bhh_ref)
    d = jnp.tanh(_mmx(hn, dw1_ref[...]) + db1_ref[...])
    d = jnp.tanh(_mmx(d, dw2_ref[...]) + db2_ref[...])
    o_ref[...] = _mmx(d, dw3_ref[...]) + db3_ref[...]


def _node_spec():
    return pl.BlockSpec((BN, 128), lambda i: (i, 0))


def _pack_spec():
    return pl.BlockSpec((BN, 64), lambda i: (i, 0))


def _packed_node_out():
    return [jax.ShapeDtypeStruct((NPAD, 128), jnp.float32),
            jax.ShapeDtypeStruct((NPAD, 64), jnp.int32),
            jax.ShapeDtypeStruct((NPAD, 64), jnp.int32)]


def _full_spec(shape):
    nd = len(shape)
    return pl.BlockSpec(shape, lambda i: (0,) * nd)


def _encode_call(x, encW, encb, w1a, b1, w1b):
    return pl.pallas_call(
        _encode_body,
        grid=(NPAD // BN,),
        in_specs=[_node_spec(), _full_spec((128, 128)), _full_spec((1, 128)),
                  _full_spec((128, 128)), _full_spec((1, 128)),
                  _full_spec((128, 128))],
        out_specs=[_node_spec(), _pack_spec(), _pack_spec()],
        out_shape=_packed_node_out(),
    )(x, encW, encb, w1a, b1, w1b)


def _edge_mlp_call(ga, gb, w2s, b2):
    espec = pl.BlockSpec((BE, 128), lambda i: (i, 0))
    return pl.pallas_call(
        _edge_mlp_body,
        grid=(EHC // BE,),
        in_specs=[espec, espec,
                  _full_spec((128, 128)), _full_spec((128, 128)),
                  _full_spec((128, 128)), _full_spec((128, 128)),
                  _full_spec((1, 128))],
        out_specs=[espec, espec],
        out_shape=[jax.ShapeDtypeStruct((EHC, 128), jnp.float32)] * 2,
    )(ga, gb, w2s[0], w2s[1], w2s[2], w2s[3], b2)


def _gru_next_call(s0, s1, s2, s3, c0, c1, c2, c3, h, w3, b3, wih, bih,
                   whh, bhh, w1a, b1, w1b):
    cspec = pl.BlockSpec((BN, 16), lambda i: (i, 0))
    return pl.pallas_call(
        _gru_next_body,
        grid=(NPAD // BN,),
        in_specs=[_node_spec(), _node_spec(), _node_spec(), _node_spec(),
                  cspec, cspec, cspec, cspec, _node_spec(),
                  _full_spec((128, 128)), _full_spec((1, 128)),
                  _full_spec((128, 384)), _full_spec((1, 384)),
                  _full_spec((128, 384)), _full_spec((1, 384)),
                  _full_spec((128, 128)), _full_spec((1, 128)),
                  _full_spec((128, 128))],
        out_specs=[_node_spec(), _pack_spec(), _pack_spec()],
        out_shape=_packed_node_out(),
    )(s0, s1, s2, s3, c0, c1, c2, c3, h, w3, b3, wih, bih, whh, bhh,
      w1a, b1, w1b)


def _gru_decode_call(s0, s1, s2, s3, c0, c1, c2, c3, h, w3, b3, wih, bih,
                     whh, bhh, dw1, db1, dw2, db2, dw3, db3):
    cspec = pl.BlockSpec((BN, 16), lambda i: (i, 0))
    return pl.pallas_call(
        _gru_decode_body,
        grid=(NPAD // BN,),
        in_specs=[_node_spec(), _node_spec(), _node_spec(), _node_spec(),
                  cspec, cspec, cspec, cspec, _node_spec(),
                  _full_spec((128, 128)), _full_spec((1, 128)),
                  _full_spec((128, 384)), _full_spec((1, 384)),
                  _full_spec((128, 384)), _full_spec((1, 384)),
                  _full_spec((128, 128)), _full_spec((1, 128)),
                  _full_spec((128, 128)), _full_spec((1, 128)),
                  _full_spec((128, 128)), _full_spec((1, 128))],
        out_specs=_node_spec(),
        out_shape=jax.ShapeDtypeStruct((NPAD, 128), jnp.float32),
    )(s0, s1, s2, s3, c0, c1, c2, c3, h, w3, b3, wih, bih, whh, bhh,
      dw1, db1, dw2, db2, dw3, db3)


def _sc_gather_cnt(a_tab, b_tab, dst_r, src_r, zeros_c, ones_w):
    @functools.partial(
        pl.kernel,
        out_type=[jax.ShapeDtypeStruct((EC, 64), jnp.int32),
                  jax.ShapeDtypeStruct((EC, 64), jnp.int32),
                  jax.ShapeDtypeStruct((2, NPAD, 16), jnp.float32)],
        mesh=_sc_mesh(),
        scratch_types=[pltpu.VMEM_SHARED((NPAD, 16), jnp.float32),
                       pltpu.VMEM((W, 16), jnp.float32)],
        compiler_params=pltpu.CompilerParams(use_tc_tiling_on_sc=False))
    def k(a_hbm, b_hbm, di_hbm, si_hbm, zc_hbm, ones_hbm,
          ga_hbm, gb_hbm, oc_hbm, c_sh, ones_v):
        core = lax.axis_index("core")
        sub = lax.axis_index("subcore")
        sl = pl.ds(sub * ROWS_PER_SUB, ROWS_PER_SUB)
        pltpu.sync_copy(zc_hbm.at[sl], c_sh.at[sl])
        pltpu.sync_copy(ones_hbm, ones_v)
        plsc.subcore_barrier()

        def body(di_v, si_v, ga_v, gb_v):
            pltpu.sync_copy(a_hbm.at[di_v.at[0]], ga_v)
            pltpu.sync_copy(b_hbm.at[si_v.at[0]], gb_v)
            pltpu.sync_copy(ones_v, c_sh.at[di_v.at[0]], add=True)

        pltpu.emit_pipeline(
            body,
            grid=(EC // W,),
            in_specs=[pl.BlockSpec((1, W), lambda i: (0, i)),
                      pl.BlockSpec((1, W), lambda i: (0, i))],
            out_specs=[pl.BlockSpec((W, 64), lambda i: (i, 0)),
                       pl.BlockSpec((W, 64), lambda i: (i, 0))],
            core_axis_name=("core", "subcore"),
            dimension_semantics=(pltpu.PARALLEL,),
        )(di_hbm, si_hbm, ga_hbm, gb_hbm)

        plsc.subcore_barrier()
        pltpu.sync_copy(c_sh.at[sl], oc_hbm.at[core, sl])

    return k(a_tab, b_tab, dst_r, src_r, zeros_c, ones_w)


def _sc_gather(a_tab, b_tab, dst_r, src_r):
    @functools.partial(
        pl.kernel,
        out_type=[jax.ShapeDtypeStruct((EC, 64), jnp.int32)] * 2,
        mesh=_sc_mesh(),
        compiler_params=pltpu.CompilerParams(use_tc_tiling_on_sc=False))
    def k(a_hbm, b_hbm, di_hbm, si_hbm, ga_hbm, gb_hbm):
        def body(di_v, si_v, ga_v, gb_v):
            pltpu.sync_copy(a_hbm.at[di_v.at[0]], ga_v)
            pltpu.sync_copy(b_hbm.at[si_v.at[0]], gb_v)

        pltpu.emit_pipeline(
            body,
            grid=(EC // W,),
            in_specs=[pl.BlockSpec((1, W), lambda i: (0, i)),
                      pl.BlockSpec((1, W), lambda i: (0, i))],
            out_specs=[pl.BlockSpec((W, 64), lambda i: (i, 0)),
                       pl.BlockSpec((W, 64), lambda i: (i, 0))],
            core_axis_name=("core", "subcore"),
            dimension_semantics=(pltpu.PARALLEL,),
        )(di_hbm, si_hbm, ga_hbm, gb_hbm)

    return k(a_tab, b_tab, dst_r, src_r)


def _sc_scatter_add(te, to, de_r, do_r, zeros_n):
    @functools.partial(
        pl.kernel,
        out_type=jax.ShapeDtypeStruct((2, NPAD, 128), jnp.float32),
        mesh=_sc_mesh(),
        scratch_types=[pltpu.VMEM_SHARED((NPAD, 128), jnp.float32)])
    def k(te_hbm, to_hbm, de_hbm, do_hbm, z_hbm, o_hbm, s_sh):
        core = lax.axis_index("core")
        sub = lax.axis_index("subcore")
        sl = pl.ds(sub * ROWS_PER_SUB, ROWS_PER_SUB)
        pltpu.sync_copy(z_hbm.at[sl], s_sh.at[sl])
        plsc.subcore_barrier()

        def body(t_v, d_v):
            pltpu.sync_copy(t_v, s_sh.at[d_v.at[0]], add=True)

        for t_hbm, d_hbm in ((te_hbm, de_hbm), (to_hbm, do_hbm)):
            pltpu.emit_pipeline(
                body,
                grid=(EHC // W,),
                in_specs=[pl.BlockSpec((W, 128), lambda i: (i, 0)),
                          pl.BlockSpec((1, W), lambda i: (0, i))],
                core_axis_name=("core", "subcore"),
                dimension_semantics=(pltpu.PARALLEL,),
            )(t_hbm, d_hbm)

        plsc.subcore_barrier()
        pltpu.sync_copy(s_sh.at[sl], o_hbm.at[core, sl])

    return k(te, to, de_r, do_r, zeros_n)


def _sc_count(dst_r, ones_w, zeros_c):
    @functools.partial(
        pl.kernel,
        out_type=jax.ShapeDtypeStruct((2, NPAD, 16), jnp.float32),
        mesh=_sc_mesh(),
        scratch_types=[pltpu.VMEM_SHARED((NPAD, 16), jnp.float32)])
    def k(di_hbm, ones_hbm, z_hbm, o_hbm, c_sh):
        core = lax.axis_index("core")
        sub = lax.axis_index("subcore")
        sl = pl.ds(sub * ROWS_PER_SUB, ROWS_PER_SUB)
        pltpu.sync_copy(z_hbm.at[sl], c_sh.at[sl])
        plsc.subcore_barrier()

        def body(ones_v, di_v):
            pltpu.sync_copy(ones_v, c_sh.at[di_v.at[0]], add=True)

        pltpu.emit_pipeline(
            body,
            grid=(EPAD // W,),
            in_specs=[pl.BlockSpec((W, 16), lambda i: (0, 0)),
                      pl.BlockSpec((1, W), lambda i: (0, i))],
            core_axis_name=("core", "subcore"),
            dimension_semantics=(pltpu.PARALLEL,),
        )(ones_hbm, di_hbm)

        plsc.subcore_barrier()
        pltpu.sync_copy(c_sh.at[sl], o_hbm.at[core, sl])

    return k(dst_r, ones_w, zeros_c)


def kernel(x, edge_index, batch, enc_W, enc_b, mW1, mb1, mW2, mb2, mW3, mb3,
           gWih, gWhh, gbih, gbhh, dW1, db1, dW2, db2, dW3, db3):
    del batch
    f32 = jnp.float32
    loops = jnp.arange(N, dtype=edge_index.dtype)
    src = jnp.concatenate([edge_index[0], loops])
    dst = jnp.concatenate([edge_index[1], loops])
    pad = jnp.full((EPAD - ETOT,), NPAD - 1, dtype=edge_index.dtype)
    src_full = jnp.concatenate([src, pad])
    dst_full = jnp.concatenate([dst, pad])
    src_r = src_full.reshape(1, EPAD)
    dst_r = dst_full.reshape(1, EPAD)
    de_r = dst_full[0::2].reshape(1, EH)
    do_r = dst_full[1::2].reshape(1, EH)

    x_pad = jnp.zeros((NPAD, x.shape[1]), f32).at[:N].set(x)
    zeros_n = jnp.zeros((NPAD, 128), f32)
    zeros_c = jnp.zeros((NPAD, 16), f32)
    ones_w = jnp.ones((W, 16), f32)

    encb_r = enc_b.reshape(1, 128)
    dw3_pad = jnp.zeros((128, 128), f32).at[:, 0:1].set(dW3)
    db3_pad = jnp.zeros((1, 128), f32).at[0, 0].set(db3[0])

    h, a_tab, b_tab = _encode_call(
        x_pad, enc_W, encb_r, mW1[0, :H], mb1[0].reshape(1, 128), mW1[0, H:])

    z64 = jnp.zeros((64, 128), f32)
    w2s = []
    for i in range(2):
        wtop, wbot = mW2[i][:64], mW2[i][64:]
        w2s.append([jnp.concatenate([wtop, z64]).astype(jnp.bfloat16),
                    jnp.concatenate([wbot, z64]).astype(jnp.bfloat16),
                    jnp.concatenate([z64, wtop]).astype(jnp.bfloat16),
                    jnp.concatenate([z64, wbot]).astype(jnp.bfloat16)])

    cnt = None
    for i in range(2):
        parts = []
        cparts = []
        for c in range(CK):
            esl = slice(c * EC, (c + 1) * EC)
            hsl = slice(c * EHC, (c + 1) * EHC)
            if i == 0:
                ga, gb, cp = _sc_gather_cnt(a_tab, b_tab, dst_r[:, esl],
                                            src_r[:, esl], zeros_c, ones_w)
                cparts.append(cp)
            else:
                ga, gb = _sc_gather(a_tab, b_tab, dst_r[:, esl], src_r[:, esl])
            te, to = _edge_mlp_call(ga.reshape(EHC, 128), gb.reshape(EHC, 128),
                                    w2s[i], mb2[i].reshape(1, 128))
            parts.append(_sc_scatter_add(te, to, de_r[:, hsl], do_r[:, hsl],
                                         zeros_n))
        if i == 0:
            cnt = (cparts[0][0], cparts[0][1], cparts[1][0], cparts[1][1])
        s00, s01 = parts[0][0], parts[0][1]
        s10, s11 = parts[1][0], parts[1][1]
        if i == 0:
            h, a_tab, b_tab = _gru_next_call(
                s00, s01, s10, s11, cnt[0], cnt[1], cnt[2], cnt[3], h,
                mW3[0], mb3[0].reshape(1, 128),
                gWih[0], gbih[0].reshape(1, 384), gWhh[0], gbhh[0].reshape(1, 384),
                mW1[1, :H], mb1[1].reshape(1, 128), mW1[1, H:])
        else:
            o = _gru_decode_call(
                s00, s01, s10, s11, cnt[0], cnt[1], cnt[2], cnt[3], h,
                mW3[1], mb3[1].reshape(1, 128),
                gWih[1], gbih[1].reshape(1, 384), gWhh[1], gbhh[1].reshape(1, 384),
                dW1, db1.reshape(1, 128), dW2, db2.reshape(1, 128),
                dw3_pad, db3_pad)

    return o[:N, 0].reshape(100, 100)


# speedup vs baseline: 1.7356x; 1.0635x over previous
import functools

import jax
import jax.numpy as jnp
from jax import lax
from jax.experimental import pallas as pl
from jax.experimental.pallas import tpu as pltpu
from jax.experimental.pallas import tpu_sc as plsc

N = 10000
NPAD = 10240
E = 160000
EPAD = 160256
H = 128
W = 128
BN = 1280
EH = EPAD // 2
CK = 2
EC = EPAD // CK
EHC = EH // CK
BE = 2504
NSUB = 16
ROWS_PER_SUB = NPAD // NSUB

@functools.cache
def _sc_mesh():
    return plsc.VectorSubcoreMesh(core_axis_name="core",
                                  subcore_axis_name="subcore")


def _mm(a, b):
    return jax.lax.dot_general(
        a.astype(jnp.bfloat16), b.astype(jnp.bfloat16), (((1,), (0,)), ((), ())),
        preferred_element_type=jnp.float32)


def _mmx(a, b):
    return jax.lax.dot_general(
        a, b, (((1,), (0,)), ((), ())),
        precision=jax.lax.Precision.HIGHEST,
        preferred_element_type=jnp.float32)


HI_MASK = -65536


def _pack_cols(v):
    lo = v[:, :64].astype(jnp.bfloat16).astype(jnp.float32)
    hi = v[:, 64:].astype(jnp.bfloat16).astype(jnp.float32)
    lo_i = jax.lax.bitcast_convert_type(lo, jnp.int32)
    hi_i = jax.lax.bitcast_convert_type(hi, jnp.int32)
    return jax.lax.shift_right_logical(lo_i, 16) | (hi_i & jnp.int32(HI_MASK))


def _unpack_lo(p):
    return jax.lax.bitcast_convert_type(jax.lax.shift_left(p, 16), jnp.float32)


def _unpack_hi(p):
    return jax.lax.bitcast_convert_type(p & jnp.int32(HI_MASK), jnp.float32)


def _encode_body(x_ref, encW_ref, encb_ref, w1a_ref, b1_ref, w1b_ref,
                 h_ref, a_ref, b_ref):
    h = jnp.tanh(_mm(x_ref[...], encW_ref[...]) + encb_ref[...])
    h_ref[...] = h
    a_ref[...] = _pack_cols(_mm(h, w1a_ref[...]) + b1_ref[...])
    b_ref[...] = _pack_cols(_mm(h, w1b_ref[...]))


def _mmbf(a, b):
    return jax.lax.dot_general(
        a, b, (((1,), (0,)), ((), ())),
        preferred_element_type=jnp.float32)


def _edge_mlp_body(ga_ref, gb_ref, w2ea_ref, w2eb_ref, w2oa_ref, w2ob_ref,
                   b2_ref, te_ref, to_ref):
    pa = ga_ref[...]
    pb = gb_ref[...]
    q = jnp.tanh(_unpack_lo(pa) + _unpack_lo(pb)).astype(jnp.bfloat16)
    r = jnp.tanh(_unpack_hi(pa) + _unpack_hi(pb)).astype(jnp.bfloat16)
    b2 = b2_ref[...]
    te = _mmbf(q, w2ea_ref[...]) + _mmbf(r, w2eb_ref[...]) + b2
    to = _mmbf(q, w2oa_ref[...]) + _mmbf(r, w2ob_ref[...]) + b2
    te_ref[...] = jnp.tanh(te)
    to_ref[...] = jnp.tanh(to)


def _gru_core(s0_ref, s1_ref, s2_ref, s3_ref, c0_ref, c1_ref, c2_ref, c3_ref,
              h_ref, cw1a_ref, cb1_ref, cw1b_ref, cw2_ref, cb2_ref,
              w3_ref, b3_ref, wih_ref, bih_ref, whh_ref, bhh_ref):
    h = h_ref[...]
    t1s = jnp.tanh((_mm(h, cw1a_ref[...]) + cb1_ref[...]) + _mm(h, cw1b_ref[...]))
    t2s = jnp.tanh(_mm(t1s, cw2_ref[...]) + cb2_ref[...])
    s = ((s0_ref[...] + s1_ref[...]) + (s2_ref[...] + s3_ref[...])) + t2s
    c = ((c0_ref[...][:, 0:1] + c1_ref[...][:, 0:1])
         + (c2_ref[...][:, 0:1] + c3_ref[...][:, 0:1])) + 1.0
    aggr = _mmx(s / c, w3_ref[...]) + b3_ref[...]
    gi = _mm(aggr, wih_ref[...]) + bih_ref[...]
    gh = _mm(h, whh_ref[...]) + bhh_ref[...]
    r = jax.nn.sigmoid(gi[:, :H] + gh[:, :H])
    z = jax.nn.sigmoid(gi[:, H:2 * H] + gh[:, H:2 * H])
    n = jnp.tanh(gi[:, 2 * H:] + r * gh[:, 2 * H:])
    return (1.0 - z) * n + z * h


def _gru_next_body(s0_ref, s1_ref, s2_ref, s3_ref, c0_ref, c1_ref, c2_ref,
                   c3_ref, h_ref, cw1a_ref, cb1_ref, cw1b_ref, cw2_ref, cb2_ref,
                   w3_ref, b3_ref, wih_ref, bih_ref, whh_ref,
                   bhh_ref, w1a_ref, b1_ref, w1b_ref, h_out_ref, a_ref, b_ref):
    hn = _gru_core(s0_ref, s1_ref, s2_ref, s3_ref, c0_ref, c1_ref, c2_ref,
                   c3_ref, h_ref, cw1a_ref, cb1_ref, cw1b_ref, cw2_ref, cb2_ref,
                   w3_ref, b3_ref, wih_ref, bih_ref, whh_ref, bhh_ref)
    h_out_ref[...] = hn
    a_ref[...] = _pack_cols(_mm(hn, w1a_ref[...]) + b1_ref[...])
    b_ref[...] = _pack_cols(_mm(hn, w1b_ref[...]))


def _gru_decode_body(s0_ref, s1_ref, s2_ref, s3_ref, c0_ref, c1_ref, c2_ref,
                     c3_ref, h_ref, cw1a_ref, cb1_ref, cw1b_ref, cw2_ref,
                     cb2_ref, w3_ref, b3_ref, wih_ref, bih_ref, whh_ref,
                     bhh_ref, dw1_ref, db1_ref, dw2_ref, db2_ref, dw3_ref,
                     db3_ref, o_ref):
    hn = _gru_core(s0_ref, s1_ref, s2_ref, s3_ref, c0_ref, c1_ref, c2_ref,
                   c3_ref, h_ref, cw1a_ref, cb1_ref, cw1b_ref, cw2_ref,
                   cb2_ref, w3_ref, b3_ref, wih_ref, bih_ref, whh_ref,
                   bhh_ref)
    d = jnp.tanh(_mmx(hn, dw1_ref[...]) + db1_ref[...])
    d = jnp.tanh(_mmx(d, dw2_ref[...]) + db2_ref[...])
    o_ref[...] = _mmx(d, dw3_ref[...]) + db3_ref[...]


def _node_spec():
    return pl.BlockSpec((BN, 128), lambda i: (i, 0))


def _pack_spec():
    return pl.BlockSpec((BN, 64), lambda i: (i, 0))


def _packed_node_out():
    return [jax.ShapeDtypeStruct((NPAD, 128), jnp.float32),
            jax.ShapeDtypeStruct((NPAD, 64), jnp.int32),
            jax.ShapeDtypeStruct((NPAD, 64), jnp.int32)]


def _full_spec(shape):
    nd = len(shape)
    return pl.BlockSpec(shape, lambda i: (0,) * nd)


def _encode_call(x, encW, encb, w1a, b1, w1b):
    return pl.pallas_call(
        _encode_body,
        grid=(NPAD // BN,),
        in_specs=[_node_spec(), _full_spec((128, 128)), _full_spec((1, 128)),
                  _full_spec((128, 128)), _full_spec((1, 128)),
                  _full_spec((128, 128))],
        out_specs=[_node_spec(), _pack_spec(), _pack_spec()],
        out_shape=_packed_node_out(),
    )(x, encW, encb, w1a, b1, w1b)


def _edge_mlp_call(ga, gb, w2s, b2):
    espec = pl.BlockSpec((BE, 128), lambda i: (i, 0))
    return pl.pallas_call(
        _edge_mlp_body,
        grid=(EHC // BE,),
        in_specs=[espec, espec,
                  _full_spec((128, 128)), _full_spec((128, 128)),
                  _full_spec((128, 128)), _full_spec((128, 128)),
                  _full_spec((1, 128))],
        out_specs=[espec, espec],
        out_shape=[jax.ShapeDtypeStruct((EHC, 128), jnp.float32)] * 2,
    )(ga, gb, w2s[0], w2s[1], w2s[2], w2s[3], b2)


def _gru_next_call(s0, s1, s2, s3, c0, c1, c2, c3, h, cw, w3, b3, wih, bih,
                   whh, bhh, w1a, b1, w1b):
    cspec = pl.BlockSpec((BN, 16), lambda i: (i, 0))
    return pl.pallas_call(
        _gru_next_body,
        grid=(NPAD // BN,),
        in_specs=[_node_spec(), _node_spec(), _node_spec(), _node_spec(),
                  cspec, cspec, cspec, cspec, _node_spec(),
                  _full_spec((128, 128)), _full_spec((1, 128)),
                  _full_spec((128, 128)), _full_spec((128, 128)),
                  _full_spec((1, 128)),
                  _full_spec((128, 128)), _full_spec((1, 128)),
                  _full_spec((128, 384)), _full_spec((1, 384)),
                  _full_spec((128, 384)), _full_spec((1, 384)),
                  _full_spec((128, 128)), _full_spec((1, 128)),
                  _full_spec((128, 128))],
        out_specs=[_node_spec(), _pack_spec(), _pack_spec()],
        out_shape=_packed_node_out(),
    )(s0, s1, s2, s3, c0, c1, c2, c3, h, cw[0], cw[1], cw[2], cw[3], cw[4],
      w3, b3, wih, bih, whh, bhh, w1a, b1, w1b)


def _gru_decode_call(s0, s1, s2, s3, c0, c1, c2, c3, h, cw, w3, b3, wih, bih,
                     whh, bhh, dw1, db1, dw2, db2, dw3, db3):
    cspec = pl.BlockSpec((BN, 16), lambda i: (i, 0))
    return pl.pallas_call(
        _gru_decode_body,
        grid=(NPAD // BN,),
        in_specs=[_node_spec(), _node_spec(), _node_spec(), _node_spec(),
                  cspec, cspec, cspec, cspec, _node_spec(),
                  _full_spec((128, 128)), _full_spec((1, 128)),
                  _full_spec((128, 128)), _full_spec((128, 128)),
                  _full_spec((1, 128)),
                  _full_spec((128, 128)), _full_spec((1, 128)),
                  _full_spec((128, 384)), _full_spec((1, 384)),
                  _full_spec((128, 384)), _full_spec((1, 384)),
                  _full_spec((128, 128)), _full_spec((1, 128)),
                  _full_spec((128, 128)), _full_spec((1, 128)),
                  _full_spec((128, 128)), _full_spec((1, 128))],
        out_specs=_node_spec(),
        out_shape=jax.ShapeDtypeStruct((NPAD, 128), jnp.float32),
    )(s0, s1, s2, s3, c0, c1, c2, c3, h, cw[0], cw[1], cw[2], cw[3], cw[4],
      w3, b3, wih, bih, whh, bhh, dw1, db1, dw2, db2, dw3, db3)


def _sc_gather_cnt(a_tab, b_tab, dst_r, src_r, zeros_c, ones_w):
    @functools.partial(
        pl.kernel,
        out_type=[jax.ShapeDtypeStruct((EC, 64), jnp.int32),
                  jax.ShapeDtypeStruct((EC, 64), jnp.int32),
                  jax.ShapeDtypeStruct((2, NPAD, 16), jnp.float32)],
        mesh=_sc_mesh(),
        scratch_types=[pltpu.VMEM_SHARED((NPAD, 16), jnp.float32),
                       pltpu.VMEM((W, 16), jnp.float32)],
        compiler_params=pltpu.CompilerParams(use_tc_tiling_on_sc=False))
    def k(a_hbm, b_hbm, di_hbm, si_hbm, zc_hbm, ones_hbm,
          ga_hbm, gb_hbm, oc_hbm, c_sh, ones_v):
        core = lax.axis_index("core")
        sub = lax.axis_index("subcore")
        sl = pl.ds(sub * ROWS_PER_SUB, ROWS_PER_SUB)
        pltpu.sync_copy(zc_hbm.at[sl], c_sh.at[sl])
        pltpu.sync_copy(ones_hbm, ones_v)
        plsc.subcore_barrier()

        def body(di_v, si_v, ga_v, gb_v):
            pltpu.sync_copy(a_hbm.at[di_v.at[0]], ga_v)
            pltpu.sync_copy(b_hbm.at[si_v.at[0]], gb_v)
            pltpu.sync_copy(ones_v, c_sh.at[di_v.at[0]], add=True)

        pltpu.emit_pipeline(
            body,
            grid=(EC // W,),
            in_specs=[pl.BlockSpec((1, W), lambda i: (0, i)),
                      pl.BlockSpec((1, W), lambda i: (0, i))],
            out_specs=[pl.BlockSpec((W, 64), lambda i: (i, 0)),
                       pl.BlockSpec((W, 64), lambda i: (i, 0))],
            core_axis_name=("core", "subcore"),
            dimension_semantics=(pltpu.PARALLEL,),
        )(di_hbm, si_hbm, ga_hbm, gb_hbm)

        plsc.subcore_barrier()
        pltpu.sync_copy(c_sh.at[sl], oc_hbm.at[core, sl])

    return k(a_tab, b_tab, dst_r, src_r, zeros_c, ones_w)


def _sc_gather(a_tab, b_tab, dst_r, src_r):
    @functools.partial(
        pl.kernel,
        out_type=[jax.ShapeDtypeStruct((EC, 64), jnp.int32)] * 2,
        mesh=_sc_mesh(),
        compiler_params=pltpu.CompilerParams(use_tc_tiling_on_sc=False))
    def k(a_hbm, b_hbm, di_hbm, si_hbm, ga_hbm, gb_hbm):
        def body(di_v, si_v, ga_v, gb_v):
            pltpu.sync_copy(a_hbm.at[di_v.at[0]], ga_v)
            pltpu.sync_copy(b_hbm.at[si_v.at[0]], gb_v)

        pltpu.emit_pipeline(
            body,
            grid=(EC // W,),
            in_specs=[pl.BlockSpec((1, W), lambda i: (0, i)),
                      pl.BlockSpec((1, W), lambda i: (0, i))],
            out_specs=[pl.BlockSpec((W, 64), lambda i: (i, 0)),
                       pl.BlockSpec((W, 64), lambda i: (i, 0))],
            core_axis_name=("core", "subcore"),
            dimension_semantics=(pltpu.PARALLEL,),
        )(di_hbm, si_hbm, ga_hbm, gb_hbm)

    return k(a_tab, b_tab, dst_r, src_r)


def _sc_scatter_add(te, to, de_r, do_r, zeros_n):
    @functools.partial(
        pl.kernel,
        out_type=jax.ShapeDtypeStruct((2, NPAD, 128), jnp.float32),
        mesh=_sc_mesh(),
        scratch_types=[pltpu.VMEM_SHARED((NPAD, 128), jnp.float32)])
    def k(te_hbm, to_hbm, de_hbm, do_hbm, z_hbm, o_hbm, s_sh):
        core = lax.axis_index("core")
        sub = lax.axis_index("subcore")
        sl = pl.ds(sub * ROWS_PER_SUB, ROWS_PER_SUB)
        pltpu.sync_copy(z_hbm.at[sl], s_sh.at[sl])
        plsc.subcore_barrier()

        def body(t_v, d_v):
            pltpu.sync_copy(t_v, s_sh.at[d_v.at[0]], add=True)

        for t_hbm, d_hbm in ((te_hbm, de_hbm), (to_hbm, do_hbm)):
            pltpu.emit_pipeline(
                body,
                grid=(EHC // W,),
                in_specs=[pl.BlockSpec((W, 128), lambda i: (i, 0)),
                          pl.BlockSpec((1, W), lambda i: (0, i))],
                core_axis_name=("core", "subcore"),
                dimension_semantics=(pltpu.PARALLEL,),
            )(t_hbm, d_hbm)

        plsc.subcore_barrier()
        pltpu.sync_copy(s_sh.at[sl], o_hbm.at[core, sl])

    return k(te, to, de_r, do_r, zeros_n)


def _sc_count(dst_r, ones_w, zeros_c):
    @functools.partial(
        pl.kernel,
        out_type=jax.ShapeDtypeStruct((2, NPAD, 16), jnp.float32),
        mesh=_sc_mesh(),
        scratch_types=[pltpu.VMEM_SHARED((NPAD, 16), jnp.float32)])
    def k(di_hbm, ones_hbm, z_hbm, o_hbm, c_sh):
        core = lax.axis_index("core")
        sub = lax.axis_index("subcore")
        sl = pl.ds(sub * ROWS_PER_SUB, ROWS_PER_SUB)
        pltpu.sync_copy(z_hbm.at[sl], c_sh.at[sl])
        plsc.subcore_barrier()

        def body(ones_v, di_v):
            pltpu.sync_copy(ones_v, c_sh.at[di_v.at[0]], add=True)

        pltpu.emit_pipeline(
            body,
            grid=(EPAD // W,),
            in_specs=[pl.BlockSpec((W, 16), lambda i: (0, 0)),
                      pl.BlockSpec((1, W), lambda i: (0, i))],
            core_axis_name=("core", "subcore"),
            dimension_semantics=(pltpu.PARALLEL,),
        )(ones_hbm, di_hbm)

        plsc.subcore_barrier()
        pltpu.sync_copy(c_sh.at[sl], o_hbm.at[core, sl])

    return k(dst_r, ones_w, zeros_c)


def kernel(x, edge_index, batch, enc_W, enc_b, mW1, mb1, mW2, mb2, mW3, mb3,
           gWih, gWhh, gbih, gbhh, dW1, db1, dW2, db2, dW3, db3):
    del batch
    f32 = jnp.float32
    src = edge_index[0]
    dst = edge_index[1]
    pad = jnp.full((EPAD - E,), NPAD - 1, dtype=edge_index.dtype)
    src_full = jnp.concatenate([src, pad])
    dst_full = jnp.concatenate([dst, pad])
    src_r = src_full.reshape(1, EPAD)
    dst_r = dst_full.reshape(1, EPAD)
    de_r = dst_full[0::2].reshape(1, EH)
    do_r = dst_full[1::2].reshape(1, EH)

    x_pad = jnp.zeros((NPAD, x.shape[1]), f32).at[:N].set(x)
    zeros_n = jnp.zeros((NPAD, 128), f32)
    zeros_c = jnp.zeros((NPAD, 16), f32)
    ones_w = jnp.ones((W, 16), f32)

    encb_r = enc_b.reshape(1, 128)
    dw3_pad = jnp.zeros((128, 128), f32).at[:, 0:1].set(dW3)
    db3_pad = jnp.zeros((1, 128), f32).at[0, 0].set(db3[0])

    h, a_tab, b_tab = _encode_call(
        x_pad, enc_W, encb_r, mW1[0, :H], mb1[0].reshape(1, 128), mW1[0, H:])

    z64 = jnp.zeros((64, 128), f32)
    w2s = []
    for i in range(2):
        wtop, wbot = mW2[i][:64], mW2[i][64:]
        w2s.append([jnp.concatenate([wtop, z64]).astype(jnp.bfloat16),
                    jnp.concatenate([wbot, z64]).astype(jnp.bfloat16),
                    jnp.concatenate([z64, wtop]).astype(jnp.bfloat16),
                    jnp.concatenate([z64, wbot]).astype(jnp.bfloat16)])

    cnt = None
    for i in range(2):
        parts = []
        cparts = []
        for c in range(CK):
            esl = slice(c * EC, (c + 1) * EC)
            hsl = slice(c * EHC, (c + 1) * EHC)
            if i == 0:
                ga, gb, cp = _sc_gather_cnt(a_tab, b_tab, dst_r[:, esl],
                                            src_r[:, esl], zeros_c, ones_w)
                cparts.append(cp)
            else:
                ga, gb = _sc_gather(a_tab, b_tab, dst_r[:, esl], src_r[:, esl])
            te, to = _edge_mlp_call(ga.reshape(EHC, 128), gb.reshape(EHC, 128),
                                    w2s[i], mb2[i].reshape(1, 128))
            parts.append(_sc_scatter_add(te, to, de_r[:, hsl], do_r[:, hsl],
                                         zeros_n))
        if i == 0:
            cnt = (cparts[0][0], cparts[0][1], cparts[1][0], cparts[1][1])
        s00, s01 = parts[0][0], parts[0][1]
        s10, s11 = parts[1][0], parts[1][1]
        if i == 0:
            cw = (mW1[0, :H], mb1[0].reshape(1, 128), mW1[0, H:],
                  mW2[0], mb2[0].reshape(1, 128))
            h, a_tab, b_tab = _gru_next_call(
                s00, s01, s10, s11, cnt[0], cnt[1], cnt[2], cnt[3], h, cw,
                mW3[0], mb3[0].reshape(1, 128),
                gWih[0], gbih[0].reshape(1, 384), gWhh[0], gbhh[0].reshape(1, 384),
                mW1[1, :H], mb1[1].reshape(1, 128), mW1[1, H:])
        else:
            cw = (mW1[1, :H], mb1[1].reshape(1, 128), mW1[1, H:],
                  mW2[1], mb2[1].reshape(1, 128))
            o = _gru_decode_call(
                s00, s01, s10, s11, cnt[0], cnt[1], cnt[2], cnt[3], h, cw,
                mW3[1], mb3[1].reshape(1, 128),
                gWih[1], gbih[1].reshape(1, 384), gWhh[1], gbhh[1].reshape(1, 384),
                dW1, db1.reshape(1, 128), dW2, db2.reshape(1, 128),
                dw3_pad, db3_pad)

    return o[:N, 0].reshape(100, 100)


# speedup vs baseline: 1.7359x; 1.0002x over previous
import functools

import jax
import jax.numpy as jnp
from jax import lax
from jax.experimental import pallas as pl
from jax.experimental.pallas import tpu as pltpu
from jax.experimental.pallas import tpu_sc as plsc

N = 10000
NPAD = 10240
E = 160000
EPAD = 160256
H = 128
W = 128
BN = 1280
EH = EPAD // 2
CK = 2
EC = EPAD // CK
EHC = EH // CK
BE = 2504
NSUB = 16
ROWS_PER_SUB = NPAD // NSUB

@functools.cache
def _sc_mesh():
    return plsc.VectorSubcoreMesh(core_axis_name="core",
                                  subcore_axis_name="subcore")


def _mm(a, b):
    return jax.lax.dot_general(
        a.astype(jnp.bfloat16), b.astype(jnp.bfloat16), (((1,), (0,)), ((), ())),
        preferred_element_type=jnp.float32)


def _mmx(a, b):
    return jax.lax.dot_general(
        a, b, (((1,), (0,)), ((), ())),
        precision=jax.lax.Precision.HIGHEST,
        preferred_element_type=jnp.float32)


HI_MASK = -65536


def _pack_cols(v):
    lo = v[:, :64].astype(jnp.bfloat16).astype(jnp.float32)
    hi = v[:, 64:].astype(jnp.bfloat16).astype(jnp.float32)
    lo_i = jax.lax.bitcast_convert_type(lo, jnp.int32)
    hi_i = jax.lax.bitcast_convert_type(hi, jnp.int32)
    return jax.lax.shift_right_logical(lo_i, 16) | (hi_i & jnp.int32(HI_MASK))


def _unpack_lo(p):
    return jax.lax.bitcast_convert_type(jax.lax.shift_left(p, 16), jnp.float32)


def _unpack_hi(p):
    return jax.lax.bitcast_convert_type(p & jnp.int32(HI_MASK), jnp.float32)


def _encode_body(x_ref, encW_ref, encb_ref, w1a_ref, b1_ref, w1b_ref,
                 h_ref, a_ref, b_ref):
    h = jnp.tanh(_mm(x_ref[...], encW_ref[...]) + encb_ref[...])
    h_ref[...] = h
    a_ref[...] = _pack_cols(_mm(h, w1a_ref[...]) + b1_ref[...])
    b_ref[...] = _pack_cols(_mm(h, w1b_ref[...]))


def _mmbf(a, b):
    return jax.lax.dot_general(
        a, b, (((1,), (0,)), ((), ())),
        preferred_element_type=jnp.float32)


def _edge_mlp_body(ga_ref, gb_ref, w2ea_ref, w2eb_ref, w2oa_ref, w2ob_ref,
                   b2_ref, te_ref, to_ref):
    pa = ga_ref[...]
    pb = gb_ref[...]
    q = jnp.tanh(_unpack_lo(pa) + _unpack_lo(pb)).astype(jnp.bfloat16)
    r = jnp.tanh(_unpack_hi(pa) + _unpack_hi(pb)).astype(jnp.bfloat16)
    b2 = b2_ref[...]
    te = _mmbf(q, w2ea_ref[...]) + _mmbf(r, w2eb_ref[...]) + b2
    to = _mmbf(q, w2oa_ref[...]) + _mmbf(r, w2ob_ref[...]) + b2
    te_ref[...] = jnp.tanh(te)
    to_ref[...] = jnp.tanh(to)


def _gru_core(s0_ref, s1_ref, s2_ref, s3_ref, c0_ref, c1_ref, c2_ref, c3_ref,
              h_ref, cw1a_ref, cb1_ref, cw1b_ref, cw2_ref, cb2_ref,
              w3_ref, b3_ref, wih_ref, bih_ref, whh_ref, bhh_ref):
    h = h_ref[...]
    t1s = jnp.tanh((_mm(h, cw1a_ref[...]) + cb1_ref[...]) + _mm(h, cw1b_ref[...]))
    t2s = jnp.tanh(_mm(t1s, cw2_ref[...]) + cb2_ref[...])
    s = ((s0_ref[...] + s1_ref[...]) + (s2_ref[...] + s3_ref[...])) + t2s
    c = ((c0_ref[...][:, 0:1] + c1_ref[...][:, 0:1])
         + (c2_ref[...][:, 0:1] + c3_ref[...][:, 0:1])) + 1.0
    aggr = _mmx(s / c, w3_ref[...]) + b3_ref[...]
    gi = _mm(aggr, wih_ref[...]) + bih_ref[...]
    gh = _mm(h, whh_ref[...]) + bhh_ref[...]
    r = jax.nn.sigmoid(gi[:, :H] + gh[:, :H])
    z = jax.nn.sigmoid(gi[:, H:2 * H] + gh[:, H:2 * H])
    n = jnp.tanh(gi[:, 2 * H:] + r * gh[:, 2 * H:])
    return (1.0 - z) * n + z * h


def _gru_next_body(s0_ref, s1_ref, s2_ref, s3_ref, c0_ref, c1_ref, c2_ref,
                   c3_ref, h_ref, cw1a_ref, cb1_ref, cw1b_ref, cw2_ref, cb2_ref,
                   w3_ref, b3_ref, wih_ref, bih_ref, whh_ref,
                   bhh_ref, w1a_ref, b1_ref, w1b_ref, h_out_ref, a_ref, b_ref):
    hn = _gru_core(s0_ref, s1_ref, s2_ref, s3_ref, c0_ref, c1_ref, c2_ref,
                   c3_ref, h_ref, cw1a_ref, cb1_ref, cw1b_ref, cw2_ref, cb2_ref,
                   w3_ref, b3_ref, wih_ref, bih_ref, whh_ref, bhh_ref)
    h_out_ref[...] = hn
    a_ref[...] = _pack_cols(_mm(hn, w1a_ref[...]) + b1_ref[...])
    b_ref[...] = _pack_cols(_mm(hn, w1b_ref[...]))


def _gru_decode_body(s0_ref, s1_ref, s2_ref, s3_ref, c0_ref, c1_ref, c2_ref,
                     c3_ref, h_ref, cw1a_ref, cb1_ref, cw1b_ref, cw2_ref,
                     cb2_ref, w3_ref, b3_ref, wih_ref, bih_ref, whh_ref,
                     bhh_ref, dw1_ref, db1_ref, dw2_ref, db2_ref, dw3_ref,
                     db3_ref, o_ref):
    hn = _gru_core(s0_ref, s1_ref, s2_ref, s3_ref, c0_ref, c1_ref, c2_ref,
                   c3_ref, h_ref, cw1a_ref, cb1_ref, cw1b_ref, cw2_ref,
                   cb2_ref, w3_ref, b3_ref, wih_ref, bih_ref, whh_ref,
                   bhh_ref)
    d = jnp.tanh(_mmx(hn, dw1_ref[...]) + db1_ref[...])
    d = jnp.tanh(_mmx(d, dw2_ref[...]) + db2_ref[...])
    o_ref[...] = _mmx(d, dw3_ref[...]) + db3_ref[...]


def _node_spec():
    return pl.BlockSpec((BN, 128), lambda i: (i, 0))


def _pack_spec():
    return pl.BlockSpec((BN, 64), lambda i: (i, 0))


def _packed_node_out():
    return [jax.ShapeDtypeStruct((NPAD, 128), jnp.float32),
            jax.ShapeDtypeStruct((NPAD, 64), jnp.int32),
            jax.ShapeDtypeStruct((NPAD, 64), jnp.int32)]


def _full_spec(shape):
    nd = len(shape)
    return pl.BlockSpec(shape, lambda i: (0,) * nd)


def _encode_call(x, encW, encb, w1a, b1, w1b):
    return pl.pallas_call(
        _encode_body,
        grid=(NPAD // BN,),
        in_specs=[_node_spec(), _full_spec((128, 128)), _full_spec((1, 128)),
                  _full_spec((128, 128)), _full_spec((1, 128)),
                  _full_spec((128, 128))],
        out_specs=[_node_spec(), _pack_spec(), _pack_spec()],
        out_shape=_packed_node_out(),
    )(x, encW, encb, w1a, b1, w1b)


def _edge_mlp_call(ga, gb, w2s, b2):
    espec = pl.BlockSpec((BE, 128), lambda i: (i, 0))
    return pl.pallas_call(
        _edge_mlp_body,
        grid=(EHC // BE,),
        in_specs=[espec, espec,
                  _full_spec((128, 128)), _full_spec((128, 128)),
                  _full_spec((128, 128)), _full_spec((128, 128)),
                  _full_spec((1, 128))],
        out_specs=[espec, espec],
        out_shape=[jax.ShapeDtypeStruct((EHC, 128), jnp.float32)] * 2,
    )(ga, gb, w2s[0], w2s[1], w2s[2], w2s[3], b2)


def _gru_next_call(s0, s1, s2, s3, c0, c1, c2, c3, h, cw, w3, b3, wih, bih,
                   whh, bhh, w1a, b1, w1b):
    cspec = pl.BlockSpec((BN, 16), lambda i: (i, 0))
    return pl.pallas_call(
        _gru_next_body,
        grid=(NPAD // BN,),
        in_specs=[_node_spec(), _node_spec(), _node_spec(), _node_spec(),
                  cspec, cspec, cspec, cspec, _node_spec(),
                  _full_spec((128, 128)), _full_spec((1, 128)),
                  _full_spec((128, 128)), _full_spec((128, 128)),
                  _full_spec((1, 128)),
                  _full_spec((128, 128)), _full_spec((1, 128)),
                  _full_spec((128, 384)), _full_spec((1, 384)),
                  _full_spec((128, 384)), _full_spec((1, 384)),
                  _full_spec((128, 128)), _full_spec((1, 128)),
                  _full_spec((128, 128))],
        out_specs=[_node_spec(), _pack_spec(), _pack_spec()],
        out_shape=_packed_node_out(),
    )(s0, s1, s2, s3, c0, c1, c2, c3, h, cw[0], cw[1], cw[2], cw[3], cw[4],
      w3, b3, wih, bih, whh, bhh, w1a, b1, w1b)


def _gru_decode_call(s0, s1, s2, s3, c0, c1, c2, c3, h, cw, w3, b3, wih, bih,
                     whh, bhh, dw1, db1, dw2, db2, dw3, db3):
    cspec = pl.BlockSpec((BN, 16), lambda i: (i, 0))
    return pl.pallas_call(
        _gru_decode_body,
        grid=(NPAD // BN,),
        in_specs=[_node_spec(), _node_spec(), _node_spec(), _node_spec(),
                  cspec, cspec, cspec, cspec, _node_spec(),
                  _full_spec((128, 128)), _full_spec((1, 128)),
                  _full_spec((128, 128)), _full_spec((128, 128)),
                  _full_spec((1, 128)),
                  _full_spec((128, 128)), _full_spec((1, 128)),
                  _full_spec((128, 384)), _full_spec((1, 384)),
                  _full_spec((128, 384)), _full_spec((1, 384)),
                  _full_spec((128, 128)), _full_spec((1, 128)),
                  _full_spec((128, 128)), _full_spec((1, 128)),
                  _full_spec((128, 128)), _full_spec((1, 128))],
        out_specs=_node_spec(),
        out_shape=jax.ShapeDtypeStruct((NPAD, 128), jnp.float32),
    )(s0, s1, s2, s3, c0, c1, c2, c3, h, cw[0], cw[1], cw[2], cw[3], cw[4],
      w3, b3, wih, bih, whh, bhh, dw1, db1, dw2, db2, dw3, db3)


def _sc_gather_cnt(a_tab, b_tab, dst_r, src_r, zeros_c, ones_w):
    @functools.partial(
        pl.kernel,
        out_type=[jax.ShapeDtypeStruct((EC, 64), jnp.int32),
                  jax.ShapeDtypeStruct((EC, 64), jnp.int32),
                  jax.ShapeDtypeStruct((2, NPAD, 16), jnp.float32)],
        mesh=_sc_mesh(),
        scratch_types=[pltpu.VMEM_SHARED((NPAD, 16), jnp.float32),
                       pltpu.VMEM((W, 16), jnp.float32)],
        compiler_params=pltpu.CompilerParams(use_tc_tiling_on_sc=False))
    def k(a_hbm, b_hbm, di_hbm, si_hbm, zc_hbm, ones_hbm,
          ga_hbm, gb_hbm, oc_hbm, c_sh, ones_v):
        core = lax.axis_index("core")
        sub = lax.axis_index("subcore")
        sl = pl.ds(sub * ROWS_PER_SUB, ROWS_PER_SUB)
        pltpu.sync_copy(zc_hbm.at[sl], c_sh.at[sl])
        pltpu.sync_copy(ones_hbm, ones_v)
        plsc.subcore_barrier()

        def body(di_v, si_v, ga_v, gb_v):
            pltpu.sync_copy(a_hbm.at[di_v.at[0]], ga_v)
            pltpu.sync_copy(b_hbm.at[si_v.at[0]], gb_v)
            pltpu.sync_copy(ones_v, c_sh.at[di_v.at[0]], add=True)

        pltpu.emit_pipeline(
            body,
            grid=(EC // W,),
            in_specs=[pl.BlockSpec((1, W), lambda i: (0, i)),
                      pl.BlockSpec((1, W), lambda i: (0, i))],
            out_specs=[pl.BlockSpec((W, 64), lambda i: (i, 0)),
                       pl.BlockSpec((W, 64), lambda i: (i, 0))],
            core_axis_name=("core", "subcore"),
            dimension_semantics=(pltpu.PARALLEL,),
        )(di_hbm, si_hbm, ga_hbm, gb_hbm)

        plsc.subcore_barrier()
        pltpu.sync_copy(c_sh.at[sl], oc_hbm.at[core, sl])

    return k(a_tab, b_tab, dst_r, src_r, zeros_c, ones_w)


def _sc_gather(a_tab, b_tab, dst_r, src_r):
    @functools.partial(
        pl.kernel,
        out_type=[jax.ShapeDtypeStruct((EC, 64), jnp.int32)] * 2,
        mesh=_sc_mesh(),
        compiler_params=pltpu.CompilerParams(use_tc_tiling_on_sc=False))
    def k(a_hbm, b_hbm, di_hbm, si_hbm, ga_hbm, gb_hbm):
        def body(di_v, si_v, ga_v, gb_v):
            pltpu.sync_copy(a_hbm.at[di_v.at[0]], ga_v)
            pltpu.sync_copy(b_hbm.at[si_v.at[0]], gb_v)

        pltpu.emit_pipeline(
            body,
            grid=(EC // W,),
            in_specs=[pl.BlockSpec((1, W), lambda i: (0, i)),
                      pl.BlockSpec((1, W), lambda i: (0, i))],
            out_specs=[pl.BlockSpec((W, 64), lambda i: (i, 0)),
                       pl.BlockSpec((W, 64), lambda i: (i, 0))],
            core_axis_name=("core", "subcore"),
            dimension_semantics=(pltpu.PARALLEL,),
        )(di_hbm, si_hbm, ga_hbm, gb_hbm)

    return k(a_tab, b_tab, dst_r, src_r)


def _sc_scatter_add(te, to, de_r, do_r, zeros_n):
    @functools.partial(
        pl.kernel,
        out_type=jax.ShapeDtypeStruct((2, NPAD, 128), jnp.float32),
        mesh=_sc_mesh(),
        scratch_types=[pltpu.VMEM_SHARED((NPAD, 128), jnp.float32)])
    def k(te_hbm, to_hbm, de_hbm, do_hbm, z_hbm, o_hbm, s_sh):
        core = lax.axis_index("core")
        sub = lax.axis_index("subcore")
        sl = pl.ds(sub * ROWS_PER_SUB, ROWS_PER_SUB)
        pltpu.sync_copy(z_hbm.at[sl], s_sh.at[sl])
        plsc.subcore_barrier()

        def body(t_v, d_v):
            pltpu.sync_copy(t_v, s_sh.at[d_v.at[0]], add=True)

        for t_hbm, d_hbm in ((te_hbm, de_hbm), (to_hbm, do_hbm)):
            pltpu.emit_pipeline(
                body,
                grid=(EHC // W,),
                in_specs=[pl.BlockSpec((W, 128), lambda i: (i, 0)),
                          pl.BlockSpec((1, W), lambda i: (0, i))],
                core_axis_name=("core", "subcore"),
                dimension_semantics=(pltpu.PARALLEL,),
            )(t_hbm, d_hbm)

        plsc.subcore_barrier()
        pltpu.sync_copy(s_sh.at[sl], o_hbm.at[core, sl])

    return k(te, to, de_r, do_r, zeros_n)


def kernel(x, edge_index, batch, enc_W, enc_b, mW1, mb1, mW2, mb2, mW3, mb3,
           gWih, gWhh, gbih, gbhh, dW1, db1, dW2, db2, dW3, db3):
    del batch
    f32 = jnp.float32
    src = edge_index[0]
    dst = edge_index[1]
    pad = jnp.full((EPAD - E,), NPAD - 1, dtype=edge_index.dtype)
    src_full = jnp.concatenate([src, pad])
    dst_full = jnp.concatenate([dst, pad])
    src_r = src_full.reshape(1, EPAD)
    dst_r = dst_full.reshape(1, EPAD)
    de_r = dst_full[0::2].reshape(1, EH)
    do_r = dst_full[1::2].reshape(1, EH)

    x_pad = jnp.zeros((NPAD, x.shape[1]), f32).at[:N].set(x)
    zeros_n = jnp.zeros((NPAD, 128), f32)
    zeros_c = jnp.zeros((NPAD, 16), f32)
    ones_w = jnp.ones((W, 16), f32)

    encb_r = enc_b.reshape(1, 128)
    dw3_pad = jnp.zeros((128, 128), f32).at[:, 0:1].set(dW3)
    db3_pad = jnp.zeros((1, 128), f32).at[0, 0].set(db3[0])

    h, a_tab, b_tab = _encode_call(
        x_pad, enc_W, encb_r, mW1[0, :H], mb1[0].reshape(1, 128), mW1[0, H:])

    z64 = jnp.zeros((64, 128), f32)
    w2s = []
    for i in range(2):
        wtop, wbot = mW2[i][:64], mW2[i][64:]
        w2s.append([jnp.concatenate([wtop, z64]).astype(jnp.bfloat16),
                    jnp.concatenate([wbot, z64]).astype(jnp.bfloat16),
                    jnp.concatenate([z64, wtop]).astype(jnp.bfloat16),
                    jnp.concatenate([z64, wbot]).astype(jnp.bfloat16)])

    cnt = None
    for i in range(2):
        parts = []
        cparts = []
        for c in range(CK):
            esl = slice(c * EC, (c + 1) * EC)
            hsl = slice(c * EHC, (c + 1) * EHC)
            if i == 0:
                ga, gb, cp = _sc_gather_cnt(a_tab, b_tab, dst_r[:, esl],
                                            src_r[:, esl], zeros_c, ones_w)
                cparts.append(cp)
            else:
                ga, gb = _sc_gather(a_tab, b_tab, dst_r[:, esl], src_r[:, esl])
            te, to = _edge_mlp_call(ga.reshape(EHC, 128), gb.reshape(EHC, 128),
                                    w2s[i], mb2[i].reshape(1, 128))
            parts.append(_sc_scatter_add(te, to, de_r[:, hsl], do_r[:, hsl],
                                         zeros_n))
        if i == 0:
            cnt = (cparts[0][0], cparts[0][1], cparts[1][0], cparts[1][1])
        s00, s01 = parts[0][0], parts[0][1]
        s10, s11 = parts[1][0], parts[1][1]
        if i == 0:
            cw = (mW1[0, :H], mb1[0].reshape(1, 128), mW1[0, H:],
                  mW2[0], mb2[0].reshape(1, 128))
            h, a_tab, b_tab = _gru_next_call(
                s00, s01, s10, s11, cnt[0], cnt[1], cnt[2], cnt[3], h, cw,
                mW3[0], mb3[0].reshape(1, 128),
                gWih[0], gbih[0].reshape(1, 384), gWhh[0], gbhh[0].reshape(1, 384),
                mW1[1, :H], mb1[1].reshape(1, 128), mW1[1, H:])
        else:
            cw = (mW1[1, :H], mb1[1].reshape(1, 128), mW1[1, H:],
                  mW2[1], mb2[1].reshape(1, 128))
            o = _gru_decode_call(
                s00, s01, s10, s11, cnt[0], cnt[1], cnt[2], cnt[3], h, cw,
                mW3[1], mb3[1].reshape(1, 128),
                gWih[1], gbih[1].reshape(1, 384), gWhh[1], gbhh[1].reshape(1, 384),
                dW1, db1.reshape(1, 128), dW2, db2.reshape(1, 128),
                dw3_pad, db3_pad)

    return o[:N, 0].reshape(100, 100)
